# Initial kernel scaffold; baseline (speedup 1.0000x reference)
import jax
import jax.numpy as jnp
from jax.experimental import pallas as pl


def kernel(x, z, edge_index, z1edge_index, z2edge_index, z3edge_index, edge_attr, pickable, params):
    raise NotImplementedError("write your pallas kernel here")


# speedup vs baseline: 14.4745x; 14.4745x over previous
import functools
import math

import jax
import jax.numpy as jnp
from jax import lax
from jax.experimental import pallas as pl
from jax.experimental.pallas import tpu as pltpu
from jax.experimental.pallas import tpu_sc as plsc

N = 100000
E = 1600000
NPICK = 16384
R = 16
NC, NS = 2, 16
NW = NC * NS
CHUNK = 1024
NCHUNKS = E // CHUNK
TAIL = E - NCHUNKS * CHUNK
CH_FULL, CH_REM = NCHUNKS // NW, NCHUNKS % NW
RPS = 6248
RPS_LAST = N - 15 * RPS
ZCH = 312
NZC = 20
SCALE = 1.0 / math.sqrt(10.0)

SG_CHUNK = 2048
SG_FULL = E // SG_CHUNK
OWN = 3128
OWN_LAST = N - 31 * OWN
ACC_ROWS = OWN + 16

_mesh = plsc.VectorSubcoreMesh(core_axis_name="c", subcore_axis_name="s",
                               num_cores=NC, num_subcores=NS)
_SC_PARAMS = pltpu.CompilerParams(needs_layout_passes=False,
                                  use_tc_tiling_on_sc=False)


def _iota16():
    return lax.iota(jnp.int32, 16)


def _full16(v):
    return jnp.full((16,), v, jnp.int32)


def _zero_rows(ref, n):
    def zr(i, _):
        ref[i] = jnp.zeros((16,), jnp.float32)
        return 0
    lax.fori_loop(0, n, zr, 0)


def _worker_ids():
    c = lax.axis_index("c")
    s = lax.axis_index("s")
    return c, s, s * NC + c


def _zero_acc(acc, zbuf, s):
    _zero_rows(zbuf, ZCH)
    base = s * RPS
    for j in range(NZC):
        pltpu.sync_copy(zbuf, acc.at[pl.ds(base + j * ZCH, ZCH)])
    t0 = base + NZC * ZCH

    @pl.when(s < 15)
    def _():
        pltpu.sync_copy(zbuf.at[pl.ds(0, RPS - NZC * ZCH)],
                        acc.at[pl.ds(t0, RPS - NZC * ZCH)])

    @pl.when(s == 15)
    def _():
        pltpu.sync_copy(zbuf.at[pl.ds(0, RPS_LAST - NZC * ZCH)],
                        acc.at[pl.ds(t0, RPS_LAST - NZC * ZCH)])


def _edge_loop(gw, body, tail_body):
    nch = CH_FULL + (gw < CH_REM).astype(jnp.int32)

    def outer(i, _):
        body(gw + i * NW)
        return 0
    lax.fori_loop(0, nch, outer, 0)

    @pl.when(gw == 31)
    def _():
        tail_body(NCHUNKS)


def _writeback(acc, out, c, s):
    plsc.subcore_barrier()
    base = c * N + s * RPS

    @pl.when(s < 15)
    def _():
        pltpu.sync_copy(acc.at[pl.ds(s * RPS, RPS)],
                        out.at[pl.ds(base, RPS)])

    @pl.when(s == 15)
    def _():
        pltpu.sync_copy(acc.at[pl.ds(15 * RPS, RPS_LAST)],
                        out.at[pl.ds(c * N + 15 * RPS, RPS_LAST)])


def _tr_body(has_e, *refs):
    if has_e:
        (src, dst, q_t, kv_t, ea_t, out,
         idx_s, idx_d, qr, kvr, er, outr, zbuf, acc, sem) = refs
    else:
        (src, dst, q_t, kv_t, out,
         idx_s, idx_d, qr, kvr, outr, zbuf, acc, sem) = refs
        er = None
    c, s, gw = _worker_ids()
    _zero_rows(outr, 128)
    _zero_acc(acc, zbuf, s)
    plsc.subcore_barrier()
    iota = _iota16()

    def make_chunk(nrows):
        def chunk(cid):
            pltpu.sync_copy(src.at[pl.ds(cid * 8, nrows)],
                            idx_s.at[pl.ds(0, nrows)])
            pltpu.sync_copy(dst.at[pl.ds(cid * 8, nrows)],
                            idx_d.at[pl.ds(0, nrows)])
            for j in range(nrows):
                cps = [pltpu.async_copy(kv_t.at[idx_s.at[j]], kvr, sem),
                       pltpu.async_copy(q_t.at[idx_d.at[j]], qr, sem)]
                if has_e:
                    cps.append(pltpu.async_copy(
                        ea_t.at[pl.ds(cid * CHUNK + j * 128, 128)], er, sem))
                for cp in cps:
                    cp.wait()

                def grp(g, _):
                    ridx = iota + g * 16
                    score = jnp.zeros((16,), jnp.float32)
                    e_cols = []
                    for d in range(10):
                        qd = plsc.load_gather(qr, [ridx, _full16(d)])
                        kd = plsc.load_gather(kvr, [ridx, _full16(d)])
                        if has_e:
                            ed = plsc.load_gather(er, [ridx, _full16(d)])
                            e_cols.append(ed)
                            kd = kd + ed
                        score = score + qd * kd
                    ee = jnp.exp(score * SCALE)
                    for d in range(10):
                        vd = plsc.load_gather(kvr, [ridx, _full16(16 + d)])
                        if has_e:
                            vd = vd + e_cols[d]
                        plsc.store_scatter(outr, [ridx, _full16(d)], ee * vd)
                    plsc.store_scatter(outr, [ridx, _full16(10)], ee)
                    return 0
                lax.fori_loop(0, 8, grp, 0)
                pltpu.sync_copy(outr, acc.at[idx_d.at[j]], add=True)
        return chunk

    _edge_loop(gw, make_chunk(8), make_chunk(4))
    _writeback(acc, out, c, s)


def _make_tr(has_e):
    scratch = [
        pltpu.VMEM((8, 128), jnp.int32),
        pltpu.VMEM((8, 128), jnp.int32),
        pltpu.VMEM((128, 16), jnp.float32),
        pltpu.VMEM((128, 32), jnp.float32),
    ]
    if has_e:
        scratch.append(pltpu.VMEM((128, 16), jnp.float32))
    scratch += [
        pltpu.VMEM((128, 16), jnp.float32),
        pltpu.VMEM((ZCH, 16), jnp.float32),
        pltpu.VMEM_SHARED((N, 16), jnp.float32),
        pltpu.SemaphoreType.DMA,
    ]
    return pl.kernel(
        functools.partial(_tr_body, has_e),
        out_type=jax.ShapeDtypeStruct((NC * N, 16), jnp.float32),
        mesh=_mesh,
        compiler_params=_SC_PARAMS,
        scratch_types=scratch,
    )


def _gat_body(src, dst, xl_t, xr_t, att_t, out,
              idx_s, idx_d, xlr, xrr, outr, attv, zbuf, acc, sem):
    c, s, gw = _worker_ids()
    _zero_rows(outr, 128)
    _zero_acc(acc, zbuf, s)
    pltpu.sync_copy(att_t, attv)
    plsc.subcore_barrier()
    iota = _iota16()
    att_cols = [plsc.load_gather(attv, [_full16(d)]) for d in range(10)]

    def make_chunk(nrows):
        def chunk(cid):
            pltpu.sync_copy(src.at[pl.ds(cid * 8, nrows)],
                            idx_s.at[pl.ds(0, nrows)])
            pltpu.sync_copy(dst.at[pl.ds(cid * 8, nrows)],
                            idx_d.at[pl.ds(0, nrows)])
            for j in range(nrows):
                cps = [pltpu.async_copy(xl_t.at[idx_s.at[j]], xlr, sem),
                       pltpu.async_copy(xr_t.at[idx_d.at[j]], xrr, sem)]
                for cp in cps:
                    cp.wait()

                def grp(g, _):
                    ridx = iota + g * 16
                    score = jnp.zeros((16,), jnp.float32)
                    l_cols = []
                    for d in range(10):
                        ld = plsc.load_gather(xlr, [ridx, _full16(d)])
                        rd = plsc.load_gather(xrr, [ridx, _full16(d)])
                        m = ld + rd
                        m = jnp.where(m >= 0.0, m, m * 0.2)
                        score = score + m * att_cols[d]
                        l_cols.append(ld)
                    ee = jnp.exp(score)
                    for d in range(10):
                        plsc.store_scatter(outr, [ridx, _full16(d)],
                                           ee * l_cols[d])
                    plsc.store_scatter(outr, [ridx, _full16(10)], ee)
                    return 0
                lax.fori_loop(0, 8, grp, 0)
                pltpu.sync_copy(outr, acc.at[idx_d.at[j]], add=True)
        return chunk

    _edge_loop(gw, make_chunk(8), make_chunk(4))
    _writeback(acc, out, c, s)


_gat_kernel = pl.kernel(
    _gat_body,
    out_type=jax.ShapeDtypeStruct((NC * N, 16), jnp.float32),
    mesh=_mesh,
    compiler_params=_SC_PARAMS,
    scratch_types=[
        pltpu.VMEM((8, 128), jnp.int32),
        pltpu.VMEM((8, 128), jnp.int32),
        pltpu.VMEM((128, 16), jnp.float32),
        pltpu.VMEM((128, 16), jnp.float32),
        pltpu.VMEM((128, 16), jnp.float32),
        pltpu.VMEM((16,), jnp.float32),
        pltpu.VMEM((ZCH, 16), jnp.float32),
        pltpu.VMEM_SHARED((N, 16), jnp.float32),
        pltpu.SemaphoreType.DMA,
    ],
)


def _gen_body(src, dst, msg_t, out, idx_s, idx_d, outr, zbuf, acc, sem):
    c, s, gw = _worker_ids()
    _zero_acc(acc, zbuf, s)
    plsc.subcore_barrier()

    def make_chunk(nrows):
        def chunk(cid):
            pltpu.sync_copy(src.at[pl.ds(cid * 8, nrows)],
                            idx_s.at[pl.ds(0, nrows)])
            pltpu.sync_copy(dst.at[pl.ds(cid * 8, nrows)],
                            idx_d.at[pl.ds(0, nrows)])
            for j in range(nrows):
                pltpu.async_copy(
                    msg_t.at[idx_s.at[j]], outr, sem).wait()
                pltpu.sync_copy(outr, acc.at[idx_d.at[j]], add=True)
        return chunk

    _edge_loop(gw, make_chunk(8), make_chunk(4))
    _writeback(acc, out, c, s)


_gen_kernel = pl.kernel(
    _gen_body,
    out_type=jax.ShapeDtypeStruct((NC * N, 16), jnp.float32),
    mesh=_mesh,
    compiler_params=_SC_PARAMS,
    scratch_types=[
        pltpu.VMEM((8, 128), jnp.int32),
        pltpu.VMEM((8, 128), jnp.int32),
        pltpu.VMEM((128, 16), jnp.float32),
        pltpu.VMEM((ZCH, 16), jnp.float32),
        pltpu.VMEM_SHARED((N, 16), jnp.float32),
        pltpu.SemaphoreType.DMA,
    ],
)


def _sage_body(src, dst, x_t, out,
               sbuf, dbuf, stag_s, stag_d, rbuf, scr_i, scr_f, acc, sem):
    c, s, gw = _worker_ids()
    lo = gw * OWN
    iota = _iota16()
    _zero_rows(acc, ACC_ROWS)

    def scan_chunk(row0, nrows):
        pltpu.sync_copy(src.at[pl.ds(row0, nrows)], sbuf.at[pl.ds(0, nrows)])
        pltpu.sync_copy(dst.at[pl.ds(row0, nrows)], dbuf.at[pl.ds(0, nrows)])
        ngroups = nrows * 8

        def grp(g, wp):
            rr = _full16(g >> 3)
            cc = (g & 7) * 16 + iota
            dv = plsc.load_gather(dbuf, [rr, cc])
            sv = plsc.load_gather(sbuf, [rr, cc])
            own = (dv >= lo) & (dv < lo + OWN)
            pc = plsc.cumsum(jnp.where(own, 1, 0))
            pos = wp + pc - 1
            plsc.store_scatter(stag_s, [pos], sv, mask=own)
            plsc.store_scatter(stag_d, [pos], dv - lo, mask=own)
            return wp + pc[15]
        wp = lax.fori_loop(0, ngroups, grp, 0)
        for j in range(8):
            pidx = wp + iota + 16 * j
            plsc.store_scatter(stag_s, [pidx], _full16(0))
            plsc.store_scatter(stag_d, [pidx], _full16(OWN))
        nb = (wp + 127) >> 7

        def batch(b, _):
            cp = pltpu.async_copy(
                x_t.at[stag_s.at[pl.ds(b * 128, 128)]], rbuf, sem)
            cp.wait()

            def g2(g, _):
                dl = plsc.load_gather(stag_d, [b * 128 + g * 16 + iota])
                srt, perm = plsc.sort_key_val(dl, iota)
                scr_i[...] = srt
                masks = []
                for sh in (1, 2, 4, 8):
                    dsh = plsc.load_gather(scr_i, [jnp.maximum(iota - sh, 0)])
                    masks.append((dsh == srt) & (iota >= sh))
                dnx = plsc.load_gather(scr_i, [jnp.minimum(iota + 1, 15)])
                last = (dnx != srt) | (iota == 15)
                rowidx = perm + g * 16
                for d in range(10):
                    v = plsc.load_gather(rbuf, [rowidx, _full16(d)])
                    for mi, sh in enumerate((1, 2, 4, 8)):
                        scr_f[...] = v
                        vsh = plsc.load_gather(
                            scr_f, [jnp.maximum(iota - sh, 0)])
                        v = jnp.where(masks[mi], jnp.maximum(v, vsh), v)
                    old = plsc.load_gather(acc, [srt, _full16(d)])
                    plsc.store_scatter(acc, [srt, _full16(d)],
                                       jnp.maximum(old, v), mask=last)
                return 0
            lax.fori_loop(0, 8, g2, 0)
            return 0
        lax.fori_loop(0, nb, batch, 0)

    def outer(i, _):
        scan_chunk(i * 16, 16)
        return 0
    lax.fori_loop(0, SG_FULL, outer, 0)
    scan_chunk(SG_FULL * 16, 4)

    @pl.when(gw < 31)
    def _():
        pltpu.sync_copy(acc.at[pl.ds(0, OWN)], out.at[pl.ds(lo, OWN)])

    @pl.when(gw == 31)
    def _():
        pltpu.sync_copy(acc.at[pl.ds(0, OWN_LAST)],
                        out.at[pl.ds(31 * OWN, OWN_LAST)])


_sage_kernel = pl.kernel(
    _sage_body,
    out_type=jax.ShapeDtypeStruct((N, 16), jnp.float32),
    mesh=_mesh,
    compiler_params=_SC_PARAMS,
    scratch_types=[
        pltpu.VMEM((16, 128), jnp.int32),
        pltpu.VMEM((16, 128), jnp.int32),
        pltpu.VMEM((SG_CHUNK + 128,), jnp.int32),
        pltpu.VMEM((SG_CHUNK + 128,), jnp.int32),
        pltpu.VMEM((128, 16), jnp.float32),
        pltpu.VMEM((16,), jnp.int32),
        pltpu.VMEM((16,), jnp.float32),
        pltpu.VMEM((ACC_ROWS, 16), jnp.float32),
        pltpu.SemaphoreType.DMA,
    ],
)


def _pick_body(pick, logit_t, out, idxb, rbuf, sem):
    c, s, gw = _worker_ids()
    pltpu.sync_copy(pick.at[pl.ds(gw * 512, 512)], idxb)
    cps = []
    for j in range(4):
        cps.append(pltpu.async_copy(
            logit_t.at[idxb.at[pl.ds(j * 128, 128)]],
            rbuf.at[pl.ds(j * 128, 128)], sem))
    for cp in cps:
        cp.wait()
    pltpu.sync_copy(rbuf, out.at[pl.ds(gw * 512, 512)])


_pick_kernel = pl.kernel(
    _pick_body,
    out_type=jax.ShapeDtypeStruct((NPICK, 16), jnp.float32),
    mesh=_mesh,
    compiler_params=_SC_PARAMS,
    scratch_types=[
        pltpu.VMEM((512,), jnp.int32),
        pltpu.VMEM((512, 16), jnp.float32),
        pltpu.SemaphoreType.DMA,
    ],
)


def _tc_map(fn, n_out, arrays, weights, block_rows):
    rows = arrays[0].shape[0]
    grid = rows // block_rows
    in_specs = [pl.BlockSpec((block_rows, a.shape[1]), lambda i: (i, 0))
                for a in arrays]
    in_specs += [pl.BlockSpec(w.shape, lambda i: (0,) * w.ndim)
                 for w in weights]
    na = len(arrays)

    def body(*refs):
        ins = [r[...] for r in refs[:na + len(weights)]]
        outs = refs[na + len(weights):]
        res = fn(*ins)
        if n_out == 1:
            res = (res,)
        for o, v in zip(outs, res):
            o[...] = v
    out_shape = [jax.ShapeDtypeStruct((rows, 16), jnp.float32)
                 for _ in range(n_out)]
    out_specs = [pl.BlockSpec((block_rows, 16), lambda i: (i, 0))
                 for _ in range(n_out)]
    res = pl.pallas_call(
        body, grid=(grid,), in_specs=in_specs,
        out_specs=out_specs if n_out > 1 else out_specs[0],
        out_shape=out_shape if n_out > 1 else out_shape[0],
    )(*arrays, *weights)
    return res


def _pad_w(p, din=16, dout=16):
    W = p["W"]
    out = jnp.zeros((din, dout), jnp.float32)
    out = out.at[:W.shape[0], :W.shape[1]].set(W)
    b = jnp.zeros((1, dout), jnp.float32)
    if "b" in p:
        b = b.at[0, :p["b"].shape[0]].set(p["b"])
    return out, b


def _colmask():
    return (lax.broadcasted_iota(jnp.int32, (1, 16), 1) < 10).astype(
        jnp.float32)


def _norm(acc2, skip):
    acc = acc2[0] + acc2[1]
    den = acc[:, 10:11]
    return acc * _colmask() / (den + 1e-16) + skip


def _leaky_self(xl, xr, att):
    m = xl + xr
    m = jnp.where(m >= 0.0, m, m * 0.2)
    es = jnp.exp(jnp.sum(m * att, axis=1, keepdims=True))
    return es


def _gat_out(acc2, xl, xr, att, bias):
    acc = acc2[0] + acc2[1]
    es = _leaky_self(xl, xr, att)
    num = acc * _colmask() + es * xl
    den = acc[:, 10:11] + es
    return jax.nn.relu(num / (den + 1e-16) + bias)


def kernel(x, z, edge_index, z1edge_index, z2edge_index, z3edge_index,
           edge_attr, pickable, params):
    p = params
    f32 = jnp.float32

    xp = jnp.zeros((N, 8), f32).at[:, :3].set(x)
    zp = jnp.zeros((N, 8), f32).at[:, :4].set(z)
    eap = jnp.zeros((E, 8), f32).at[:, :6].set(edge_attr)
    exs = edge_index[0].reshape(E // 128, 128)
    exd = edge_index[1].reshape(E // 128, 128)
    z1s = z1edge_index[0].reshape(E // 128, 128)
    z1d = z1edge_index[1].reshape(E // 128, 128)
    z2s = z2edge_index[0].reshape(E // 128, 128)
    z2d = z2edge_index[1].reshape(E // 128, 128)
    z3s = z3edge_index[0].reshape(E // 128, 128)
    z3d = z3edge_index[1].reshape(E // 128, 128)

    Wex, bex = _pad_w(p["encx"], 8)
    Wez, bez = _pad_w(p["encz"], 8)
    We1, be1 = _pad_w(p["edge1"], 8)
    We2, be2 = _pad_w(p["edge2"])
    tr_w = {}
    for name in ("convx1", "convx2", "convz1", "convz2"):
        tp = p[name]
        tr_w[name] = {k: _pad_w(tp[k]) for k in tp}
    _, bxz = _pad_w(p["linxz"], 32)
    Wxz_t = jnp.zeros((16, 16), f32).at[:10, :10].set(p["linxz"]["W"][:10])
    Wxz_b = jnp.zeros((16, 16), f32).at[:10, :10].set(p["linxz"]["W"][10:])
    Wmsg, bmsg = _pad_w(p["convxz1"]["msg"])
    gat_w = {}
    for name in ("convxz2", "convxz3", "convxz5"):
        gp = p[name]
        Wl, bl = _pad_w(gp["l"])
        Wr, br = _pad_w(gp["r"])
        att = jnp.zeros((1, 16), f32).at[0, :10].set(gp["att"])
        bias = jnp.zeros((1, 16), f32).at[0, :10].set(gp["bias"])
        gat_w[name] = (Wl, bl, Wr, br, att, bias)
    Wl4, bl4 = _pad_w(p["convxz4"]["l"])
    Wr4, _ = _pad_w(p["convxz4"]["r"])
    Wlin, blin = _pad_w(p["lin"])

    def f_enc(xb, zb, Wx, bx, Wz, bz, Wq, bq, Wk, bk, Wv, bv,
              Wqz, bqz, Wkz, bkz, Wvz, bvz):
        x0 = jax.nn.relu(xb @ Wx + bx)
        z0 = zb @ Wz + bz
        return (x0, z0, x0 @ Wq + bq, x0 @ Wk + bk, x0 @ Wv + bv,
                z0 @ Wqz + bqz, z0 @ Wkz + bkz, z0 @ Wvz + bvz)
    tx = tr_w["convx1"]
    tz = tr_w["convz1"]
    x0, z0, qx1, kx1, vx1, qz1, kz1, vz1 = _tc_map(
        f_enc, 8, [xp, zp],
        [Wex, bex, Wez, bez, tx["q"][0], tx["q"][1], tx["k"][0], tx["k"][1],
         tx["v"][0], tx["v"][1], tz["q"][0], tz["q"][1], tz["k"][0],
         tz["k"][1], tz["v"][0], tz["v"][1]], 2000)
    kvx1 = jnp.concatenate([kx1, vx1], axis=1)
    kvz1 = jnp.concatenate([kz1, vz1], axis=1)

    def f_ea(e, W1, b1, W2, b2):
        return (e @ W1 + b1) @ W2 + b2
    ea = _tc_map(f_ea, 1, [eap], [We1, be1, We2, be2], 2000)

    tr_e = _make_tr(True)
    tr_ne = _make_tr(False)
    accx1 = tr_e(exs, exd, qx1, kvx1, ea)
    accz1 = tr_ne(z1s, z1d, qz1, kvz1)

    def f_comb_prep(a0, a1, xin, Ws, bs, Wq, bq, Wk, bk, Wv, bv):
        xn = jax.nn.relu(_norm((a0, a1), xin @ Ws + bs))
        return xn, xn @ Wq + bq, xn @ Wk + bk, xn @ Wv + bv
    tx2 = tr_w["convx2"]
    x1, qx2, kx2, vx2 = _tc_map(
        f_comb_prep, 4, [accx1[:N], accx1[N:], x0],
        [tx["skip"][0], tx["skip"][1], tx2["q"][0], tx2["q"][1],
         tx2["k"][0], tx2["k"][1], tx2["v"][0], tx2["v"][1]], 2000)
    kvx2 = jnp.concatenate([kx2, vx2], axis=1)
    accx2 = tr_e(exs, exd, qx2, kvx2, ea)

    tz2 = tr_w["convz2"]
    z1f, qz2, kz2, vz2 = _tc_map(
        f_comb_prep, 4, [accz1[:N], accz1[N:], z0],
        [tz["skip"][0], tz["skip"][1], tz2["q"][0], tz2["q"][1],
         tz2["k"][0], tz2["k"][1], tz2["v"][0], tz2["v"][1]], 2000)
    kvz2 = jnp.concatenate([kz2, vz2], axis=1)
    accz2 = tr_ne(z1s, z1d, qz2, kvz2)

    def f_xz(ax0, ax1, xin, az0, az1, zin, Wsx, bsx, Wsz, bsz,
             Wt, Wb, bxzv, Wm, bm):
        x2 = _norm((ax0, ax1), xin @ Wsx + bsx)
        z2 = _norm((az0, az1), zin @ Wsz + bsz)
        h0 = x2 @ Wt + z2 @ Wb + bxzv
        return h0, h0 @ Wm + bm
    h0, msg = _tc_map(
        f_xz, 2, [accx2[:N], accx2[N:], x1, accz2[:N], accz2[N:], z1f],
        [tx2["skip"][0], tx2["skip"][1], tz2["skip"][0], tz2["skip"][1],
         Wxz_t, Wxz_b, bxz, Wmsg, bmsg], 2000)

    accg = _gen_kernel(z1s, z1d, msg)

    def f_gen(a0, a1, hin, Wl, bl, Wr, br):
        h1 = jax.nn.relu(a0 + a1 + hin)
        return h1 @ Wl + bl, h1 @ Wr + br
    g2w = gat_w["convxz2"]
    xl2, xr2 = _tc_map(f_gen, 2, [accg[:N], accg[N:], h0],
                       [g2w[0], g2w[1], g2w[2], g2w[3]], 2000)

    att2_v = jnp.zeros((16,), f32).at[:10].set(p["convxz2"]["att"])
    accg2 = _gat_kernel(z2s, z2d, xl2, xr2, att2_v)

    def f_gat_comb(a0, a1, xl, xr, att, bias, Wl, bl, Wr, br):
        h = _gat_out((a0, a1), xl, xr, att, bias)
        return h @ Wl + bl, h @ Wr + br
    g3w = gat_w["convxz3"]
    xl3, xr3 = _tc_map(
        f_gat_comb, 2, [accg2[:N], accg2[N:], xl2, xr2],
        [g2w[4], g2w[5], g3w[0], g3w[1], g3w[2], g3w[3]], 2000)

    att3_v = jnp.zeros((16,), f32).at[:10].set(p["convxz3"]["att"])
    accg3 = _gat_kernel(z1s, z1d, xl3, xr3, att3_v)

    def f_gat3(a0, a1, xl, xr, att, bias):
        return _gat_out((a0, a1), xl, xr, att, bias)
    g3 = gat_w["convxz3"]
    h3 = _tc_map(f_gat3, 1, [accg3[:N], accg3[N:], xl3, xr3],
                 [g3[4], g3[5]], 2000)

    maxacc = _sage_kernel(z3s, z3d, h3)

    def f_sage(agg, hin, Wl, bl, Wr, Wl5, bl5, Wr5, br5):
        h4 = jax.nn.relu(agg @ Wl + bl + hin @ Wr)
        return h4 @ Wl5 + bl5, h4 @ Wr5 + br5
    g5w = gat_w["convxz5"]
    xl5, xr5 = _tc_map(f_sage, 2, [maxacc, h3],
                       [Wl4, bl4, Wr4, g5w[0], g5w[1], g5w[2], g5w[3]], 2000)

    att5_v = jnp.zeros((16,), f32).at[:10].set(p["convxz5"]["att"])
    accg5 = _gat_kernel(z1s, z1d, xl5, xr5, att5_v)

    def f_final(a0, a1, xl, xr, att, bias, Wf, bf):
        h5 = _gat_out((a0, a1), xl, xr, att, bias)
        return h5 @ Wf + bf
    logits = _tc_map(f_final, 1, [accg5[:N], accg5[N:], xl5, xr5],
                     [g5w[4], g5w[5], Wlin, blin], 2000)

    rows = _pick_kernel(pickable, logits)

    def f_soft(r):
        l0, l1 = r[:, 0:1], r[:, 1:2]
        m = jnp.maximum(l0, l1)
        e0 = jnp.exp(l0 - m)
        e1 = jnp.exp(l1 - m)
        s = e0 + e1
        return jnp.concatenate(
            [e0 / s, e1 / s, jnp.zeros((r.shape[0], 14), f32)], axis=1)
    soft = _tc_map(f_soft, 1, [rows], [], 2048)
    return soft[:, :2]


# speedup vs baseline: 18.0016x; 1.2437x over previous
import functools
import math

import jax
import jax.numpy as jnp
from jax import lax
from jax.experimental import pallas as pl
from jax.experimental.pallas import tpu as pltpu
from jax.experimental.pallas import tpu_sc as plsc

N = 100000
E = 1600000
NPICK = 16384
R = 16
NC, NS = 2, 16
NW = NC * NS
CHUNK = 1024
NCHUNKS = E // CHUNK
TAIL = E - NCHUNKS * CHUNK
CH_FULL, CH_REM = NCHUNKS // NW, NCHUNKS % NW
RPS = 6248
RPS_LAST = N - 15 * RPS
ZCH = 312
NZC = 20
SCALE = 1.0 / math.sqrt(10.0)

SG_CHUNK = 2048
SG_FULL = E // SG_CHUNK
SG_HALF = 391
OWN = 6256
OWN_LAST = N - 15 * OWN
ACC_ROWS = OWN + 16

_mesh = plsc.VectorSubcoreMesh(core_axis_name="c", subcore_axis_name="s",
                               num_cores=NC, num_subcores=NS)
_SC_PARAMS = pltpu.CompilerParams(needs_layout_passes=False,
                                  use_tc_tiling_on_sc=False)


def _iota16():
    return lax.iota(jnp.int32, 16)


def _full16(v):
    return jnp.full((16,), v, jnp.int32)


def _zero_rows(ref, n):
    def zr(i, _):
        ref[i] = jnp.zeros((16,), jnp.float32)
        return 0
    lax.fori_loop(0, n, zr, 0)


def _worker_ids():
    c = lax.axis_index("c")
    s = lax.axis_index("s")
    return c, s, s * NC + c


def _zero_acc(acc, zbuf, s):
    _zero_rows(zbuf, ZCH)
    base = s * RPS
    for j in range(NZC):
        pltpu.sync_copy(zbuf, acc.at[pl.ds(base + j * ZCH, ZCH)])
    t0 = base + NZC * ZCH

    @pl.when(s < 15)
    def _():
        pltpu.sync_copy(zbuf.at[pl.ds(0, RPS - NZC * ZCH)],
                        acc.at[pl.ds(t0, RPS - NZC * ZCH)])

    @pl.when(s == 15)
    def _():
        pltpu.sync_copy(zbuf.at[pl.ds(0, RPS_LAST - NZC * ZCH)],
                        acc.at[pl.ds(t0, RPS_LAST - NZC * ZCH)])


def _edge_loop(gw, body, tail_body):
    nch = CH_FULL + (gw < CH_REM).astype(jnp.int32)

    def outer(i, _):
        body(gw + i * NW)
        return 0
    lax.fori_loop(0, nch, outer, 0)

    @pl.when(gw == 31)
    def _():
        tail_body(NCHUNKS)


def _writeback(acc, out, c, s):
    plsc.subcore_barrier()
    base = c * N + s * RPS

    @pl.when(s < 15)
    def _():
        pltpu.sync_copy(acc.at[pl.ds(s * RPS, RPS)],
                        out.at[pl.ds(base, RPS)])

    @pl.when(s == 15)
    def _():
        pltpu.sync_copy(acc.at[pl.ds(15 * RPS, RPS_LAST)],
                        out.at[pl.ds(c * N + 15 * RPS, RPS_LAST)])


def _tr_body(has_e, *refs):
    if has_e:
        (src, dst, q_t, kv_t, ea_t, out,
         idx_s, idx_d, qr, kvr, er, outr, zbuf, acc, sem) = refs
    else:
        (src, dst, q_t, kv_t, out,
         idx_s, idx_d, qr, kvr, outr, zbuf, acc, sem) = refs
        er = None
    c, s, gw = _worker_ids()
    _zero_rows(outr, 128)
    _zero_acc(acc, zbuf, s)
    plsc.subcore_barrier()
    iota = _iota16()

    def make_chunk(nrows):
        def chunk(cid):
            pltpu.sync_copy(src.at[pl.ds(cid * 8, nrows)],
                            idx_s.at[pl.ds(0, nrows)])
            pltpu.sync_copy(dst.at[pl.ds(cid * 8, nrows)],
                            idx_d.at[pl.ds(0, nrows)])
            for j in range(nrows):
                cps = [pltpu.async_copy(kv_t.at[idx_s.at[j]], kvr, sem),
                       pltpu.async_copy(q_t.at[idx_d.at[j]], qr, sem)]
                if has_e:
                    cps.append(pltpu.async_copy(
                        ea_t.at[pl.ds(cid * CHUNK + j * 128, 128)], er, sem))
                for cp in cps:
                    cp.wait()

                def grp(g, _):
                    ridx = iota + g * 16
                    score = jnp.zeros((16,), jnp.float32)
                    e_cols = []
                    for d in range(10):
                        qd = plsc.load_gather(qr, [ridx, _full16(d)])
                        kd = plsc.load_gather(kvr, [ridx, _full16(d)])
                        if has_e:
                            ed = plsc.load_gather(er, [ridx, _full16(d)])
                            e_cols.append(ed)
                            kd = kd + ed
                        score = score + qd * kd
                    ee = jnp.exp(score * SCALE)
                    for d in range(10):
                        vd = plsc.load_gather(kvr, [ridx, _full16(16 + d)])
                        if has_e:
                            vd = vd + e_cols[d]
                        plsc.store_scatter(outr, [ridx, _full16(d)], ee * vd)
                    plsc.store_scatter(outr, [ridx, _full16(10)], ee)
                    return 0
                lax.fori_loop(0, 8, grp, 0)
                pltpu.sync_copy(outr, acc.at[idx_d.at[j]], add=True)
        return chunk

    _edge_loop(gw, make_chunk(8), make_chunk(4))
    _writeback(acc, out, c, s)


def _make_tr(has_e):
    scratch = [
        pltpu.VMEM((8, 128), jnp.int32),
        pltpu.VMEM((8, 128), jnp.int32),
        pltpu.VMEM((128, 16), jnp.float32),
        pltpu.VMEM((128, 32), jnp.float32),
    ]
    if has_e:
        scratch.append(pltpu.VMEM((128, 16), jnp.float32))
    scratch += [
        pltpu.VMEM((128, 16), jnp.float32),
        pltpu.VMEM((ZCH, 16), jnp.float32),
        pltpu.VMEM_SHARED((N, 16), jnp.float32),
        pltpu.SemaphoreType.DMA,
    ]
    return pl.kernel(
        functools.partial(_tr_body, has_e),
        out_type=jax.ShapeDtypeStruct((NC * N, 16), jnp.float32),
        mesh=_mesh,
        compiler_params=_SC_PARAMS,
        scratch_types=scratch,
    )


def _gat_body(src, dst, xl_t, xr_t, att_t, out,
              idx_s, idx_d, xlr, xrr, outr, attv, zbuf, acc, sem):
    c, s, gw = _worker_ids()
    _zero_rows(outr, 128)
    _zero_acc(acc, zbuf, s)
    pltpu.sync_copy(att_t, attv)
    plsc.subcore_barrier()
    iota = _iota16()
    att_cols = [plsc.load_gather(attv, [_full16(d)]) for d in range(10)]

    def make_chunk(nrows):
        def chunk(cid):
            pltpu.sync_copy(src.at[pl.ds(cid * 8, nrows)],
                            idx_s.at[pl.ds(0, nrows)])
            pltpu.sync_copy(dst.at[pl.ds(cid * 8, nrows)],
                            idx_d.at[pl.ds(0, nrows)])
            for j in range(nrows):
                cps = [pltpu.async_copy(xl_t.at[idx_s.at[j]], xlr, sem),
                       pltpu.async_copy(xr_t.at[idx_d.at[j]], xrr, sem)]
                for cp in cps:
                    cp.wait()

                def grp(g, _):
                    ridx = iota + g * 16
                    score = jnp.zeros((16,), jnp.float32)
                    l_cols = []
                    for d in range(10):
                        ld = plsc.load_gather(xlr, [ridx, _full16(d)])
                        rd = plsc.load_gather(xrr, [ridx, _full16(d)])
                        m = ld + rd
                        m = jnp.where(m >= 0.0, m, m * 0.2)
                        score = score + m * att_cols[d]
                        l_cols.append(ld)
                    ee = jnp.exp(score)
                    for d in range(10):
                        plsc.store_scatter(outr, [ridx, _full16(d)],
                                           ee * l_cols[d])
                    plsc.store_scatter(outr, [ridx, _full16(10)], ee)
                    return 0
                lax.fori_loop(0, 8, grp, 0)
                pltpu.sync_copy(outr, acc.at[idx_d.at[j]], add=True)
        return chunk

    _edge_loop(gw, make_chunk(8), make_chunk(4))
    _writeback(acc, out, c, s)


_gat_kernel = pl.kernel(
    _gat_body,
    out_type=jax.ShapeDtypeStruct((NC * N, 16), jnp.float32),
    mesh=_mesh,
    compiler_params=_SC_PARAMS,
    scratch_types=[
        pltpu.VMEM((8, 128), jnp.int32),
        pltpu.VMEM((8, 128), jnp.int32),
        pltpu.VMEM((128, 16), jnp.float32),
        pltpu.VMEM((128, 16), jnp.float32),
        pltpu.VMEM((128, 16), jnp.float32),
        pltpu.VMEM((16,), jnp.float32),
        pltpu.VMEM((ZCH, 16), jnp.float32),
        pltpu.VMEM_SHARED((N, 16), jnp.float32),
        pltpu.SemaphoreType.DMA,
    ],
)


def _gen_body(src, dst, msg_t, out, idx_s, idx_d, outr, zbuf, acc, sem):
    c, s, gw = _worker_ids()
    _zero_acc(acc, zbuf, s)
    plsc.subcore_barrier()

    def make_chunk(nrows):
        def chunk(cid):
            pltpu.sync_copy(src.at[pl.ds(cid * 8, nrows)],
                            idx_s.at[pl.ds(0, nrows)])
            pltpu.sync_copy(dst.at[pl.ds(cid * 8, nrows)],
                            idx_d.at[pl.ds(0, nrows)])
            for j in range(nrows):
                pltpu.async_copy(
                    msg_t.at[idx_s.at[j]], outr, sem).wait()
                pltpu.sync_copy(outr, acc.at[idx_d.at[j]], add=True)
        return chunk

    _edge_loop(gw, make_chunk(8), make_chunk(4))
    _writeback(acc, out, c, s)


_gen_kernel = pl.kernel(
    _gen_body,
    out_type=jax.ShapeDtypeStruct((NC * N, 16), jnp.float32),
    mesh=_mesh,
    compiler_params=_SC_PARAMS,
    scratch_types=[
        pltpu.VMEM((8, 128), jnp.int32),
        pltpu.VMEM((8, 128), jnp.int32),
        pltpu.VMEM((128, 16), jnp.float32),
        pltpu.VMEM((ZCH, 16), jnp.float32),
        pltpu.VMEM_SHARED((N, 16), jnp.float32),
        pltpu.SemaphoreType.DMA,
    ],
)


def _sage_body(src, dst, x_t, out,
               sbuf, dbuf, stag_s, stag_d, rbuf, acc, sem):
    c, s, gw = _worker_ids()
    lo = s * OWN
    iota = _iota16()
    _zero_rows(acc, ACC_ROWS)

    def scan_chunk(row0, nrows):
        cps = [pltpu.async_copy(src.at[pl.ds(row0, nrows)],
                                sbuf.at[pl.ds(0, nrows)], sem),
               pltpu.async_copy(dst.at[pl.ds(row0, nrows)],
                                dbuf.at[pl.ds(0, nrows)], sem)]
        for cp in cps:
            cp.wait()
        ngroups = nrows * 8

        def grp(g, wp):
            rr = _full16(g >> 3)
            cc = (g & 7) * 16 + iota
            dv = plsc.load_gather(dbuf, [rr, cc])
            sv = plsc.load_gather(sbuf, [rr, cc])
            own = (dv >= lo) & (dv < lo + OWN)
            pc = plsc.cumsum(jnp.where(own, 1, 0))
            pos = wp + pc - 1
            plsc.store_scatter(stag_s, [pos], sv, mask=own)
            plsc.store_scatter(stag_d, [pos], dv - lo, mask=own)
            return wp + pc[15]
        wp = lax.fori_loop(0, ngroups, grp, 0)
        for j in range(8):
            pidx = wp + iota + 16 * j
            plsc.store_scatter(stag_s, [pidx], _full16(0))
            plsc.store_scatter(stag_d, [pidx], _full16(OWN))
        nb = (wp + 127) >> 7

        def batch(b, _):
            cp = pltpu.async_copy(
                x_t.at[stag_s.at[pl.ds(b * 128, 128)]], rbuf, sem)
            cp.wait()

            def g2(g, _):
                dl = plsc.load_gather(stag_d, [b * 128 + g * 16 + iota])
                for l in range(16):
                    di = _full16(dl[l])
                    row = plsc.load_gather(rbuf, [_full16(g * 16 + l), iota])
                    old = plsc.load_gather(acc, [di, iota])
                    plsc.store_scatter(acc, [di, iota],
                                       jnp.maximum(old, row))
                return 0
            lax.fori_loop(0, 8, g2, 0)
            return 0
        lax.fori_loop(0, nb, batch, 0)

    half = jnp.where(c == 0, 0, SG_HALF)
    nfull = jnp.where(c == 0, SG_HALF, SG_FULL - SG_HALF)

    def outer(i, _):
        scan_chunk((half + i) * 16, 16)
        return 0
    lax.fori_loop(0, nfull, outer, 0)

    @pl.when(c == 1)
    def _():
        scan_chunk(SG_FULL * 16, 4)
    base = c * N + lo

    @pl.when(s < 15)
    def _():
        pltpu.sync_copy(acc.at[pl.ds(0, OWN)], out.at[pl.ds(base, OWN)])

    @pl.when(s == 15)
    def _():
        pltpu.sync_copy(acc.at[pl.ds(0, OWN_LAST)],
                        out.at[pl.ds(base, OWN_LAST)])


_sage_kernel = pl.kernel(
    _sage_body,
    out_type=jax.ShapeDtypeStruct((NC * N, 16), jnp.float32),
    mesh=_mesh,
    compiler_params=_SC_PARAMS,
    scratch_types=[
        pltpu.VMEM((16, 128), jnp.int32),
        pltpu.VMEM((16, 128), jnp.int32),
        pltpu.VMEM((SG_CHUNK + 128,), jnp.int32),
        pltpu.VMEM((SG_CHUNK + 128,), jnp.int32),
        pltpu.VMEM((128, 16), jnp.float32),
        pltpu.VMEM((ACC_ROWS, 16), jnp.float32),
        pltpu.SemaphoreType.DMA,
    ],
)


def _pick_body(pick, logit_t, out, idxb, rbuf, sem):
    c, s, gw = _worker_ids()
    pltpu.sync_copy(pick.at[pl.ds(gw * 512, 512)], idxb)
    cps = []
    for j in range(4):
        cps.append(pltpu.async_copy(
            logit_t.at[idxb.at[pl.ds(j * 128, 128)]],
            rbuf.at[pl.ds(j * 128, 128)], sem))
    for cp in cps:
        cp.wait()
    pltpu.sync_copy(rbuf, out.at[pl.ds(gw * 512, 512)])


_pick_kernel = pl.kernel(
    _pick_body,
    out_type=jax.ShapeDtypeStruct((NPICK, 16), jnp.float32),
    mesh=_mesh,
    compiler_params=_SC_PARAMS,
    scratch_types=[
        pltpu.VMEM((512,), jnp.int32),
        pltpu.VMEM((512, 16), jnp.float32),
        pltpu.SemaphoreType.DMA,
    ],
)


def _tc_map(fn, n_out, arrays, weights, block_rows):
    rows = arrays[0].shape[0]
    grid = rows // block_rows
    in_specs = [pl.BlockSpec((block_rows, a.shape[1]), lambda i: (i, 0))
                for a in arrays]
    in_specs += [pl.BlockSpec(w.shape, lambda i: (0,) * w.ndim)
                 for w in weights]
    na = len(arrays)

    def body(*refs):
        ins = [r[...] for r in refs[:na + len(weights)]]
        outs = refs[na + len(weights):]
        res = fn(*ins)
        if n_out == 1:
            res = (res,)
        for o, v in zip(outs, res):
            o[...] = v
    out_shape = [jax.ShapeDtypeStruct((rows, 16), jnp.float32)
                 for _ in range(n_out)]
    out_specs = [pl.BlockSpec((block_rows, 16), lambda i: (i, 0))
                 for _ in range(n_out)]
    res = pl.pallas_call(
        body, grid=(grid,), in_specs=in_specs,
        out_specs=out_specs if n_out > 1 else out_specs[0],
        out_shape=out_shape if n_out > 1 else out_shape[0],
    )(*arrays, *weights)
    return res


def _pad_w(p, din=16, dout=16):
    W = p["W"]
    out = jnp.zeros((din, dout), jnp.float32)
    out = out.at[:W.shape[0], :W.shape[1]].set(W)
    b = jnp.zeros((1, dout), jnp.float32)
    if "b" in p:
        b = b.at[0, :p["b"].shape[0]].set(p["b"])
    return out, b


def _colmask():
    return (lax.broadcasted_iota(jnp.int32, (1, 16), 1) < 10).astype(
        jnp.float32)


def _norm(acc2, skip):
    acc = acc2[0] + acc2[1]
    den = acc[:, 10:11]
    return acc * _colmask() / (den + 1e-16) + skip


def _leaky_self(xl, xr, att):
    m = xl + xr
    m = jnp.where(m >= 0.0, m, m * 0.2)
    es = jnp.exp(jnp.sum(m * att, axis=1, keepdims=True))
    return es


def _gat_out(acc2, xl, xr, att, bias):
    acc = acc2[0] + acc2[1]
    es = _leaky_self(xl, xr, att)
    num = acc * _colmask() + es * xl
    den = acc[:, 10:11] + es
    return jax.nn.relu(num / (den + 1e-16) + bias)


def kernel(x, z, edge_index, z1edge_index, z2edge_index, z3edge_index,
           edge_attr, pickable, params):
    p = params
    f32 = jnp.float32

    xp = jnp.zeros((N, 8), f32).at[:, :3].set(x)
    zp = jnp.zeros((N, 8), f32).at[:, :4].set(z)
    eap = jnp.zeros((E, 8), f32).at[:, :6].set(edge_attr)
    exs = edge_index[0].reshape(E // 128, 128)
    exd = edge_index[1].reshape(E // 128, 128)
    z1s = z1edge_index[0].reshape(E // 128, 128)
    z1d = z1edge_index[1].reshape(E // 128, 128)
    z2s = z2edge_index[0].reshape(E // 128, 128)
    z2d = z2edge_index[1].reshape(E // 128, 128)
    z3s = z3edge_index[0].reshape(E // 128, 128)
    z3d = z3edge_index[1].reshape(E // 128, 128)

    Wex, bex = _pad_w(p["encx"], 8)
    Wez, bez = _pad_w(p["encz"], 8)
    We1, be1 = _pad_w(p["edge1"], 8)
    We2, be2 = _pad_w(p["edge2"])
    tr_w = {}
    for name in ("convx1", "convx2", "convz1", "convz2"):
        tp = p[name]
        tr_w[name] = {k: _pad_w(tp[k]) for k in tp}
    _, bxz = _pad_w(p["linxz"], 32)
    Wxz_t = jnp.zeros((16, 16), f32).at[:10, :10].set(p["linxz"]["W"][:10])
    Wxz_b = jnp.zeros((16, 16), f32).at[:10, :10].set(p["linxz"]["W"][10:])
    Wmsg, bmsg = _pad_w(p["convxz1"]["msg"])
    gat_w = {}
    for name in ("convxz2", "convxz3", "convxz5"):
        gp = p[name]
        Wl, bl = _pad_w(gp["l"])
        Wr, br = _pad_w(gp["r"])
        att = jnp.zeros((1, 16), f32).at[0, :10].set(gp["att"])
        bias = jnp.zeros((1, 16), f32).at[0, :10].set(gp["bias"])
        gat_w[name] = (Wl, bl, Wr, br, att, bias)
    Wl4, bl4 = _pad_w(p["convxz4"]["l"])
    Wr4, _ = _pad_w(p["convxz4"]["r"])
    Wlin, blin = _pad_w(p["lin"])

    def f_enc(xb, zb, Wx, bx, Wz, bz, Wq, bq, Wk, bk, Wv, bv,
              Wqz, bqz, Wkz, bkz, Wvz, bvz):
        x0 = jax.nn.relu(xb @ Wx + bx)
        z0 = zb @ Wz + bz
        return (x0, z0, x0 @ Wq + bq, x0 @ Wk + bk, x0 @ Wv + bv,
                z0 @ Wqz + bqz, z0 @ Wkz + bkz, z0 @ Wvz + bvz)
    tx = tr_w["convx1"]
    tz = tr_w["convz1"]
    x0, z0, qx1, kx1, vx1, qz1, kz1, vz1 = _tc_map(
        f_enc, 8, [xp, zp],
        [Wex, bex, Wez, bez, tx["q"][0], tx["q"][1], tx["k"][0], tx["k"][1],
         tx["v"][0], tx["v"][1], tz["q"][0], tz["q"][1], tz["k"][0],
         tz["k"][1], tz["v"][0], tz["v"][1]], 2000)
    kvx1 = jnp.concatenate([kx1, vx1], axis=1)
    kvz1 = jnp.concatenate([kz1, vz1], axis=1)

    def f_ea(e, W1, b1, W2, b2):
        return (e @ W1 + b1) @ W2 + b2
    ea = _tc_map(f_ea, 1, [eap], [We1, be1, We2, be2], 2000)

    tr_e = _make_tr(True)
    tr_ne = _make_tr(False)
    accx1 = tr_e(exs, exd, qx1, kvx1, ea)
    accz1 = tr_ne(z1s, z1d, qz1, kvz1)

    def f_comb_prep(a0, a1, xin, Ws, bs, Wq, bq, Wk, bk, Wv, bv):
        xn = jax.nn.relu(_norm((a0, a1), xin @ Ws + bs))
        return xn, xn @ Wq + bq, xn @ Wk + bk, xn @ Wv + bv
    tx2 = tr_w["convx2"]
    x1, qx2, kx2, vx2 = _tc_map(
        f_comb_prep, 4, [accx1[:N], accx1[N:], x0],
        [tx["skip"][0], tx["skip"][1], tx2["q"][0], tx2["q"][1],
         tx2["k"][0], tx2["k"][1], tx2["v"][0], tx2["v"][1]], 2000)
    kvx2 = jnp.concatenate([kx2, vx2], axis=1)
    accx2 = tr_e(exs, exd, qx2, kvx2, ea)

    tz2 = tr_w["convz2"]
    z1f, qz2, kz2, vz2 = _tc_map(
        f_comb_prep, 4, [accz1[:N], accz1[N:], z0],
        [tz["skip"][0], tz["skip"][1], tz2["q"][0], tz2["q"][1],
         tz2["k"][0], tz2["k"][1], tz2["v"][0], tz2["v"][1]], 2000)
    kvz2 = jnp.concatenate([kz2, vz2], axis=1)
    accz2 = tr_ne(z1s, z1d, qz2, kvz2)

    def f_xz(ax0, ax1, xin, az0, az1, zin, Wsx, bsx, Wsz, bsz,
             Wt, Wb, bxzv, Wm, bm):
        x2 = _norm((ax0, ax1), xin @ Wsx + bsx)
        z2 = _norm((az0, az1), zin @ Wsz + bsz)
        h0 = x2 @ Wt + z2 @ Wb + bxzv
        return h0, h0 @ Wm + bm
    h0, msg = _tc_map(
        f_xz, 2, [accx2[:N], accx2[N:], x1, accz2[:N], accz2[N:], z1f],
        [tx2["skip"][0], tx2["skip"][1], tz2["skip"][0], tz2["skip"][1],
         Wxz_t, Wxz_b, bxz, Wmsg, bmsg], 2000)

    accg = _gen_kernel(z1s, z1d, msg)

    def f_gen(a0, a1, hin, Wl, bl, Wr, br):
        h1 = jax.nn.relu(a0 + a1 + hin)
        return h1 @ Wl + bl, h1 @ Wr + br
    g2w = gat_w["convxz2"]
    xl2, xr2 = _tc_map(f_gen, 2, [accg[:N], accg[N:], h0],
                       [g2w[0], g2w[1], g2w[2], g2w[3]], 2000)

    att2_v = jnp.zeros((16,), f32).at[:10].set(p["convxz2"]["att"])
    accg2 = _gat_kernel(z2s, z2d, xl2, xr2, att2_v)

    def f_gat_comb(a0, a1, xl, xr, att, bias, Wl, bl, Wr, br):
        h = _gat_out((a0, a1), xl, xr, att, bias)
        return h @ Wl + bl, h @ Wr + br
    g3w = gat_w["convxz3"]
    xl3, xr3 = _tc_map(
        f_gat_comb, 2, [accg2[:N], accg2[N:], xl2, xr2],
        [g2w[4], g2w[5], g3w[0], g3w[1], g3w[2], g3w[3]], 2000)

    att3_v = jnp.zeros((16,), f32).at[:10].set(p["convxz3"]["att"])
    accg3 = _gat_kernel(z1s, z1d, xl3, xr3, att3_v)

    def f_gat3(a0, a1, xl, xr, att, bias):
        return _gat_out((a0, a1), xl, xr, att, bias)
    g3 = gat_w["convxz3"]
    h3 = _tc_map(f_gat3, 1, [accg3[:N], accg3[N:], xl3, xr3],
                 [g3[4], g3[5]], 2000)

    maxacc = _sage_kernel(z3s, z3d, h3)

    def f_sage(m0, m1, hin, Wl, bl, Wr, Wl5, bl5, Wr5, br5):
        agg = jnp.maximum(m0, m1)
        h4 = jax.nn.relu(agg @ Wl + bl + hin @ Wr)
        return h4 @ Wl5 + bl5, h4 @ Wr5 + br5
    g5w = gat_w["convxz5"]
    xl5, xr5 = _tc_map(f_sage, 2, [maxacc[:N], maxacc[N:], h3],
                       [Wl4, bl4, Wr4, g5w[0], g5w[1], g5w[2], g5w[3]], 2000)

    att5_v = jnp.zeros((16,), f32).at[:10].set(p["convxz5"]["att"])
    accg5 = _gat_kernel(z1s, z1d, xl5, xr5, att5_v)

    def f_final(a0, a1, xl, xr, att, bias, Wf, bf):
        h5 = _gat_out((a0, a1), xl, xr, att, bias)
        return h5 @ Wf + bf
    logits = _tc_map(f_final, 1, [accg5[:N], accg5[N:], xl5, xr5],
                     [g5w[4], g5w[5], Wlin, blin], 2000)

    rows = _pick_kernel(pickable, logits)

    def f_soft(r):
        l0, l1 = r[:, 0:1], r[:, 1:2]
        m = jnp.maximum(l0, l1)
        e0 = jnp.exp(l0 - m)
        e1 = jnp.exp(l1 - m)
        s = e0 + e1
        return jnp.concatenate(
            [e0 / s, e1 / s, jnp.zeros((r.shape[0], 14), f32)], axis=1)
    soft = _tc_map(f_soft, 1, [rows], [], 2048)
    return soft[:, :2]


# speedup vs baseline: 20.8428x; 1.1578x over previous
import functools
import math

import jax
import jax.numpy as jnp
from jax import lax
from jax.experimental import pallas as pl
from jax.experimental.pallas import tpu as pltpu
from jax.experimental.pallas import tpu_sc as plsc

N = 100000
E = 1600000
NPICK = 16384
R = 16
NC, NS = 2, 16
NW = NC * NS
CHUNK = 1024
NCHUNKS = E // CHUNK
TAIL = E - NCHUNKS * CHUNK
CH_FULL, CH_REM = NCHUNKS // NW, NCHUNKS % NW
RPS = 6248
RPS_LAST = N - 15 * RPS
ZCH = 104
NZC = 60
SCALE = 1.0 / math.sqrt(10.0)

SG_CHUNK = 2048
SG_FULL = E // SG_CHUNK
SG_HALF = 391
OWN = 6256
OWN_LAST = N - 15 * OWN
ACC_ROWS = OWN + 16

_mesh = plsc.VectorSubcoreMesh(core_axis_name="c", subcore_axis_name="s",
                               num_cores=NC, num_subcores=NS)
_SC_PARAMS = pltpu.CompilerParams(needs_layout_passes=False,
                                  use_tc_tiling_on_sc=False)


def _iota16():
    return lax.iota(jnp.int32, 16)


def _full16(v):
    return jnp.full((16,), v, jnp.int32)


def _zero_rows(ref, n):
    def zr(i, _):
        ref[i] = jnp.zeros((16,), jnp.float32)
        return 0
    lax.fori_loop(0, n, zr, 0)


def _worker_ids():
    c = lax.axis_index("c")
    s = lax.axis_index("s")
    return c, s, s * NC + c


def _zero_acc(acc, zbuf, s):
    _zero_rows(zbuf, ZCH)
    base = s * RPS
    for j in range(NZC):
        pltpu.sync_copy(zbuf, acc.at[pl.ds(base + j * ZCH, ZCH)])
    t0 = base + NZC * ZCH

    @pl.when(s < 15)
    def _():
        pltpu.sync_copy(zbuf.at[pl.ds(0, RPS - NZC * ZCH)],
                        acc.at[pl.ds(t0, RPS - NZC * ZCH)])

    @pl.when(s == 15)
    def _():
        pltpu.sync_copy(zbuf.at[pl.ds(0, RPS_LAST - NZC * ZCH)],
                        acc.at[pl.ds(t0, RPS_LAST - NZC * ZCH)])


def _edge_loop(gw, body, tail_body):
    nch = CH_FULL + (gw < CH_REM).astype(jnp.int32)

    def outer(i, _):
        body(gw + i * NW)
        return 0
    lax.fori_loop(0, nch, outer, 0)

    @pl.when(gw == 31)
    def _():
        tail_body(NCHUNKS)


def _writeback(acc, out, c, s):
    plsc.subcore_barrier()
    base = c * N + s * RPS

    @pl.when(s < 15)
    def _():
        pltpu.sync_copy(acc.at[pl.ds(s * RPS, RPS)],
                        out.at[pl.ds(base, RPS)])

    @pl.when(s == 15)
    def _():
        pltpu.sync_copy(acc.at[pl.ds(15 * RPS, RPS_LAST)],
                        out.at[pl.ds(c * N + 15 * RPS, RPS_LAST)])


def _tr_body(has_e, *refs):
    if has_e:
        (src, dst, q_t, kv_t, ea_t, out, idx_s, idx_d,
         qr0, qr1, kvr0, kvr1, er0, er1, outr0, outr1,
         zbuf, acc, sg0, sg1, ss0, ss1) = refs
        er = [er0, er1]
    else:
        (src, dst, q_t, kv_t, out, idx_s, idx_d,
         qr0, qr1, kvr0, kvr1, outr0, outr1,
         zbuf, acc, sg0, sg1, ss0, ss1) = refs
        er = [None, None]
    qr, kvr, outr = [qr0, qr1], [kvr0, kvr1], [outr0, outr1]
    sg, ss = [sg0, sg1], [ss0, ss1]
    c, s, gw = _worker_ids()
    _zero_rows(outr0, 128)
    _zero_rows(outr1, 128)
    _zero_acc(acc, zbuf, s)
    plsc.subcore_barrier()
    iota = _iota16()

    def make_chunk(nrows):
        def chunk(cid):
            icps = [pltpu.async_copy(src.at[pl.ds(cid * 8, nrows)],
                                     idx_s.at[pl.ds(0, nrows)], sg0),
                    pltpu.async_copy(dst.at[pl.ds(cid * 8, nrows)],
                                     idx_d.at[pl.ds(0, nrows)], sg1)]
            for cp in icps:
                cp.wait()

            def fire(j):
                b = j & 1
                cps = [pltpu.async_copy(kv_t.at[idx_s.at[j]], kvr[b], sg[b]),
                       pltpu.async_copy(q_t.at[idx_d.at[j]], qr[b], sg[b])]
                if has_e:
                    cps.append(pltpu.async_copy(
                        ea_t.at[pl.ds(cid * CHUNK + j * 128, 128)],
                        er[b], sg[b]))
                return cps

            def compute(b):
                def grp(g, _):
                    ridx = iota + g * 16
                    score = jnp.zeros((16,), jnp.float32)
                    e_cols = []
                    for d in range(10):
                        qd = plsc.load_gather(qr[b], [ridx, _full16(d)])
                        kd = plsc.load_gather(kvr[b], [ridx, _full16(d)])
                        if has_e:
                            ed = plsc.load_gather(er[b], [ridx, _full16(d)])
                            e_cols.append(ed)
                            kd = kd + ed
                        score = score + qd * kd
                    ee = jnp.exp(score * SCALE)
                    for d in range(10):
                        vd = plsc.load_gather(kvr[b], [ridx, _full16(16 + d)])
                        if has_e:
                            vd = vd + e_cols[d]
                        plsc.store_scatter(outr[b], [ridx, _full16(d)],
                                           ee * vd)
                    plsc.store_scatter(outr[b], [ridx, _full16(10)], ee)
                    return 0
                lax.fori_loop(0, 8, grp, 0)

            gh = {0: fire(0)}
            sh = {}
            for j in range(nrows):
                b = j & 1
                for cp in gh.pop(j):
                    cp.wait()
                if j + 1 < nrows:
                    gh[j + 1] = fire(j + 1)
                if j - 2 in sh:
                    sh.pop(j - 2).wait()
                compute(b)
                sh[j] = pltpu.async_copy(outr[b], acc.at[idx_d.at[j]],
                                         ss[b], add=True)
            for h in sh.values():
                h.wait()
        return chunk

    _edge_loop(gw, make_chunk(8), make_chunk(4))
    _writeback(acc, out, c, s)


def _make_tr(has_e):
    scratch = [
        pltpu.VMEM((8, 128), jnp.int32),
        pltpu.VMEM((8, 128), jnp.int32),
        pltpu.VMEM((128, 16), jnp.float32),
        pltpu.VMEM((128, 16), jnp.float32),
        pltpu.VMEM((128, 32), jnp.float32),
        pltpu.VMEM((128, 32), jnp.float32),
    ]
    if has_e:
        scratch += [pltpu.VMEM((128, 16), jnp.float32),
                    pltpu.VMEM((128, 16), jnp.float32)]
    scratch += [
        pltpu.VMEM((128, 16), jnp.float32),
        pltpu.VMEM((128, 16), jnp.float32),
        pltpu.VMEM((ZCH, 16), jnp.float32),
        pltpu.VMEM_SHARED((N, 16), jnp.float32),
        pltpu.SemaphoreType.DMA,
        pltpu.SemaphoreType.DMA,
        pltpu.SemaphoreType.DMA,
        pltpu.SemaphoreType.DMA,
    ]
    return pl.kernel(
        functools.partial(_tr_body, has_e),
        out_type=jax.ShapeDtypeStruct((NC * N, 16), jnp.float32),
        mesh=_mesh,
        compiler_params=_SC_PARAMS,
        scratch_types=scratch,
    )


def _gat_body(src, dst, xl_t, xr_t, att_t, out,
              idx_s, idx_d, xlr0, xlr1, xrr0, xrr1, outr0, outr1,
              attv, zbuf, acc, sg0, sg1, ss0, ss1):
    xlr, xrr, outr = [xlr0, xlr1], [xrr0, xrr1], [outr0, outr1]
    sg, ss = [sg0, sg1], [ss0, ss1]
    c, s, gw = _worker_ids()
    _zero_rows(outr0, 128)
    _zero_rows(outr1, 128)
    _zero_acc(acc, zbuf, s)
    pltpu.sync_copy(att_t, attv)
    plsc.subcore_barrier()
    iota = _iota16()
    att_cols = [plsc.load_gather(attv, [_full16(d)]) for d in range(10)]

    def make_chunk(nrows):
        def chunk(cid):
            icps = [pltpu.async_copy(src.at[pl.ds(cid * 8, nrows)],
                                     idx_s.at[pl.ds(0, nrows)], sg0),
                    pltpu.async_copy(dst.at[pl.ds(cid * 8, nrows)],
                                     idx_d.at[pl.ds(0, nrows)], sg1)]
            for cp in icps:
                cp.wait()

            def fire(j):
                b = j & 1
                return [pltpu.async_copy(xl_t.at[idx_s.at[j]], xlr[b],
                                         sg[b]),
                        pltpu.async_copy(xr_t.at[idx_d.at[j]], xrr[b],
                                         sg[b])]

            def compute(b):
                def grp(g, _):
                    ridx = iota + g * 16
                    score = jnp.zeros((16,), jnp.float32)
                    l_cols = []
                    for d in range(10):
                        ld = plsc.load_gather(xlr[b], [ridx, _full16(d)])
                        rd = plsc.load_gather(xrr[b], [ridx, _full16(d)])
                        m = ld + rd
                        m = jnp.where(m >= 0.0, m, m * 0.2)
                        score = score + m * att_cols[d]
                        l_cols.append(ld)
                    ee = jnp.exp(score)
                    for d in range(10):
                        plsc.store_scatter(outr[b], [ridx, _full16(d)],
                                           ee * l_cols[d])
                    plsc.store_scatter(outr[b], [ridx, _full16(10)], ee)
                    return 0
                lax.fori_loop(0, 8, grp, 0)

            gh = {0: fire(0)}
            sh = {}
            for j in range(nrows):
                b = j & 1
                for cp in gh.pop(j):
                    cp.wait()
                if j + 1 < nrows:
                    gh[j + 1] = fire(j + 1)
                if j - 2 in sh:
                    sh.pop(j - 2).wait()
                compute(b)
                sh[j] = pltpu.async_copy(outr[b], acc.at[idx_d.at[j]],
                                         ss[b], add=True)
            for h in sh.values():
                h.wait()
        return chunk

    _edge_loop(gw, make_chunk(8), make_chunk(4))
    _writeback(acc, out, c, s)


_gat_kernel = pl.kernel(
    _gat_body,
    out_type=jax.ShapeDtypeStruct((NC * N, 16), jnp.float32),
    mesh=_mesh,
    compiler_params=_SC_PARAMS,
    scratch_types=[
        pltpu.VMEM((8, 128), jnp.int32),
        pltpu.VMEM((8, 128), jnp.int32),
        pltpu.VMEM((128, 16), jnp.float32),
        pltpu.VMEM((128, 16), jnp.float32),
        pltpu.VMEM((128, 16), jnp.float32),
        pltpu.VMEM((128, 16), jnp.float32),
        pltpu.VMEM((128, 16), jnp.float32),
        pltpu.VMEM((128, 16), jnp.float32),
        pltpu.VMEM((16,), jnp.float32),
        pltpu.VMEM((ZCH, 16), jnp.float32),
        pltpu.VMEM_SHARED((N, 16), jnp.float32),
        pltpu.SemaphoreType.DMA,
        pltpu.SemaphoreType.DMA,
        pltpu.SemaphoreType.DMA,
        pltpu.SemaphoreType.DMA,
    ],
)


def _gen_body(src, dst, msg_t, out, idx_s, idx_d, outr0, outr1,
              zbuf, acc, sg0, sg1, ss0, ss1):
    outr = [outr0, outr1]
    sg, ss = [sg0, sg1], [ss0, ss1]
    c, s, gw = _worker_ids()
    _zero_acc(acc, zbuf, s)
    plsc.subcore_barrier()

    def make_chunk(nrows):
        def chunk(cid):
            icps = [pltpu.async_copy(src.at[pl.ds(cid * 8, nrows)],
                                     idx_s.at[pl.ds(0, nrows)], sg0),
                    pltpu.async_copy(dst.at[pl.ds(cid * 8, nrows)],
                                     idx_d.at[pl.ds(0, nrows)], sg1)]
            for cp in icps:
                cp.wait()

            def fire(j):
                b = j & 1
                return pltpu.async_copy(msg_t.at[idx_s.at[j]], outr[b],
                                        sg[b])

            gh = {0: fire(0)}
            sh = {}
            for j in range(nrows):
                b = j & 1
                gh.pop(j).wait()
                sh[j] = pltpu.async_copy(outr[b], acc.at[idx_d.at[j]],
                                         ss[b], add=True)
                if j + 1 < nrows:
                    if j - 1 in sh:
                        sh.pop(j - 1).wait()
                    gh[j + 1] = fire(j + 1)
            for h in sh.values():
                h.wait()
        return chunk

    _edge_loop(gw, make_chunk(8), make_chunk(4))
    _writeback(acc, out, c, s)


_gen_kernel = pl.kernel(
    _gen_body,
    out_type=jax.ShapeDtypeStruct((NC * N, 16), jnp.float32),
    mesh=_mesh,
    compiler_params=_SC_PARAMS,
    scratch_types=[
        pltpu.VMEM((8, 128), jnp.int32),
        pltpu.VMEM((8, 128), jnp.int32),
        pltpu.VMEM((128, 16), jnp.float32),
        pltpu.VMEM((128, 16), jnp.float32),
        pltpu.VMEM((ZCH, 16), jnp.float32),
        pltpu.VMEM_SHARED((N, 16), jnp.float32),
        pltpu.SemaphoreType.DMA,
        pltpu.SemaphoreType.DMA,
        pltpu.SemaphoreType.DMA,
        pltpu.SemaphoreType.DMA,
    ],
)


def _sage_body(src, dst, x_t, out,
               sbuf, dbuf, stag_s, stag_d, rbuf, acc, sem):
    c, s, gw = _worker_ids()
    lo = s * OWN
    iota = _iota16()
    _zero_rows(acc, ACC_ROWS)

    def scan_chunk(row0, nrows):
        cps = [pltpu.async_copy(src.at[pl.ds(row0, nrows)],
                                sbuf.at[pl.ds(0, nrows)], sem),
               pltpu.async_copy(dst.at[pl.ds(row0, nrows)],
                                dbuf.at[pl.ds(0, nrows)], sem)]
        for cp in cps:
            cp.wait()
        ngroups = nrows * 8

        def grp(g, wp):
            rr = _full16(g >> 3)
            cc = (g & 7) * 16 + iota
            dv = plsc.load_gather(dbuf, [rr, cc])
            sv = plsc.load_gather(sbuf, [rr, cc])
            own = (dv >= lo) & (dv < lo + OWN)
            pc = plsc.cumsum(jnp.where(own, 1, 0))
            pos = wp + pc - 1
            plsc.store_scatter(stag_s, [pos], sv, mask=own)
            plsc.store_scatter(stag_d, [pos], dv - lo, mask=own)
            return wp + pc[15]
        wp = lax.fori_loop(0, ngroups, grp, 0)
        for j in range(8):
            pidx = wp + iota + 16 * j
            plsc.store_scatter(stag_s, [pidx], _full16(0))
            plsc.store_scatter(stag_d, [pidx], _full16(OWN))
        nb = (wp + 127) >> 7

        def batch(b, _):
            cp = pltpu.async_copy(
                x_t.at[stag_s.at[pl.ds(b * 128, 128)]], rbuf, sem)
            cp.wait()

            def g2(g, _):
                dl = plsc.load_gather(stag_d, [b * 128 + g * 16 + iota])
                for l in range(16):
                    di = _full16(dl[l])
                    row = plsc.load_gather(rbuf, [_full16(g * 16 + l), iota])
                    old = plsc.load_gather(acc, [di, iota])
                    plsc.store_scatter(acc, [di, iota],
                                       jnp.maximum(old, row))
                return 0
            lax.fori_loop(0, 8, g2, 0)
            return 0
        lax.fori_loop(0, nb, batch, 0)

    half = jnp.where(c == 0, 0, SG_HALF)
    nfull = jnp.where(c == 0, SG_HALF, SG_FULL - SG_HALF)

    def outer(i, _):
        scan_chunk((half + i) * 16, 16)
        return 0
    lax.fori_loop(0, nfull, outer, 0)

    @pl.when(c == 1)
    def _():
        scan_chunk(SG_FULL * 16, 4)
    base = c * N + lo

    @pl.when(s < 15)
    def _():
        pltpu.sync_copy(acc.at[pl.ds(0, OWN)], out.at[pl.ds(base, OWN)])

    @pl.when(s == 15)
    def _():
        pltpu.sync_copy(acc.at[pl.ds(0, OWN_LAST)],
                        out.at[pl.ds(base, OWN_LAST)])


_sage_kernel = pl.kernel(
    _sage_body,
    out_type=jax.ShapeDtypeStruct((NC * N, 16), jnp.float32),
    mesh=_mesh,
    compiler_params=_SC_PARAMS,
    scratch_types=[
        pltpu.VMEM((16, 128), jnp.int32),
        pltpu.VMEM((16, 128), jnp.int32),
        pltpu.VMEM((SG_CHUNK + 128,), jnp.int32),
        pltpu.VMEM((SG_CHUNK + 128,), jnp.int32),
        pltpu.VMEM((128, 16), jnp.float32),
        pltpu.VMEM((ACC_ROWS, 16), jnp.float32),
        pltpu.SemaphoreType.DMA,
    ],
)


def _pick_body(pick, logit_t, out, idxb, rbuf, sem):
    c, s, gw = _worker_ids()
    pltpu.sync_copy(pick.at[pl.ds(gw * 512, 512)], idxb)
    cps = []
    for j in range(4):
        cps.append(pltpu.async_copy(
            logit_t.at[idxb.at[pl.ds(j * 128, 128)]],
            rbuf.at[pl.ds(j * 128, 128)], sem))
    for cp in cps:
        cp.wait()
    pltpu.sync_copy(rbuf, out.at[pl.ds(gw * 512, 512)])


_pick_kernel = pl.kernel(
    _pick_body,
    out_type=jax.ShapeDtypeStruct((NPICK, 16), jnp.float32),
    mesh=_mesh,
    compiler_params=_SC_PARAMS,
    scratch_types=[
        pltpu.VMEM((512,), jnp.int32),
        pltpu.VMEM((512, 16), jnp.float32),
        pltpu.SemaphoreType.DMA,
    ],
)


def _tc_map(fn, n_out, arrays, weights, block_rows):
    rows = arrays[0].shape[0]
    grid = rows // block_rows
    in_specs = [pl.BlockSpec((block_rows, a.shape[1]), lambda i: (i, 0))
                for a in arrays]
    in_specs += [pl.BlockSpec(w.shape, lambda i: (0,) * w.ndim)
                 for w in weights]
    na = len(arrays)

    def body(*refs):
        ins = [r[...] for r in refs[:na + len(weights)]]
        outs = refs[na + len(weights):]
        res = fn(*ins)
        if n_out == 1:
            res = (res,)
        for o, v in zip(outs, res):
            o[...] = v
    out_shape = [jax.ShapeDtypeStruct((rows, 16), jnp.float32)
                 for _ in range(n_out)]
    out_specs = [pl.BlockSpec((block_rows, 16), lambda i: (i, 0))
                 for _ in range(n_out)]
    res = pl.pallas_call(
        body, grid=(grid,), in_specs=in_specs,
        out_specs=out_specs if n_out > 1 else out_specs[0],
        out_shape=out_shape if n_out > 1 else out_shape[0],
    )(*arrays, *weights)
    return res


def _pad_w(p, din=16, dout=16):
    W = p["W"]
    out = jnp.zeros((din, dout), jnp.float32)
    out = out.at[:W.shape[0], :W.shape[1]].set(W)
    b = jnp.zeros((1, dout), jnp.float32)
    if "b" in p:
        b = b.at[0, :p["b"].shape[0]].set(p["b"])
    return out, b


def _colmask():
    return (lax.broadcasted_iota(jnp.int32, (1, 16), 1) < 10).astype(
        jnp.float32)


def _norm(acc2, skip):
    acc = acc2[0] + acc2[1]
    den = acc[:, 10:11]
    return acc * _colmask() / (den + 1e-16) + skip


def _leaky_self(xl, xr, att):
    m = xl + xr
    m = jnp.where(m >= 0.0, m, m * 0.2)
    es = jnp.exp(jnp.sum(m * att, axis=1, keepdims=True))
    return es


def _gat_out(acc2, xl, xr, att, bias):
    acc = acc2[0] + acc2[1]
    es = _leaky_self(xl, xr, att)
    num = acc * _colmask() + es * xl
    den = acc[:, 10:11] + es
    return jax.nn.relu(num / (den + 1e-16) + bias)


def kernel(x, z, edge_index, z1edge_index, z2edge_index, z3edge_index,
           edge_attr, pickable, params):
    p = params
    f32 = jnp.float32

    xp = jnp.zeros((N, 8), f32).at[:, :3].set(x)
    zp = jnp.zeros((N, 8), f32).at[:, :4].set(z)
    eap = jnp.zeros((E, 8), f32).at[:, :6].set(edge_attr)
    exs = edge_index[0].reshape(E // 128, 128)
    exd = edge_index[1].reshape(E // 128, 128)
    z1s = z1edge_index[0].reshape(E // 128, 128)
    z1d = z1edge_index[1].reshape(E // 128, 128)
    z2s = z2edge_index[0].reshape(E // 128, 128)
    z2d = z2edge_index[1].reshape(E // 128, 128)
    z3s = z3edge_index[0].reshape(E // 128, 128)
    z3d = z3edge_index[1].reshape(E // 128, 128)

    Wex, bex = _pad_w(p["encx"], 8)
    Wez, bez = _pad_w(p["encz"], 8)
    We1, be1 = _pad_w(p["edge1"], 8)
    We2, be2 = _pad_w(p["edge2"])
    tr_w = {}
    for name in ("convx1", "convx2", "convz1", "convz2"):
        tp = p[name]
        tr_w[name] = {k: _pad_w(tp[k]) for k in tp}
    _, bxz = _pad_w(p["linxz"], 32)
    Wxz_t = jnp.zeros((16, 16), f32).at[:10, :10].set(p["linxz"]["W"][:10])
    Wxz_b = jnp.zeros((16, 16), f32).at[:10, :10].set(p["linxz"]["W"][10:])
    Wmsg, bmsg = _pad_w(p["convxz1"]["msg"])
    gat_w = {}
    for name in ("convxz2", "convxz3", "convxz5"):
        gp = p[name]
        Wl, bl = _pad_w(gp["l"])
        Wr, br = _pad_w(gp["r"])
        att = jnp.zeros((1, 16), f32).at[0, :10].set(gp["att"])
        bias = jnp.zeros((1, 16), f32).at[0, :10].set(gp["bias"])
        gat_w[name] = (Wl, bl, Wr, br, att, bias)
    Wl4, bl4 = _pad_w(p["convxz4"]["l"])
    Wr4, _ = _pad_w(p["convxz4"]["r"])
    Wlin, blin = _pad_w(p["lin"])

    def f_enc(xb, zb, Wx, bx, Wz, bz, Wq, bq, Wk, bk, Wv, bv,
              Wqz, bqz, Wkz, bkz, Wvz, bvz):
        x0 = jax.nn.relu(xb @ Wx + bx)
        z0 = zb @ Wz + bz
        return (x0, z0, x0 @ Wq + bq, x0 @ Wk + bk, x0 @ Wv + bv,
                z0 @ Wqz + bqz, z0 @ Wkz + bkz, z0 @ Wvz + bvz)
    tx = tr_w["convx1"]
    tz = tr_w["convz1"]
    x0, z0, qx1, kx1, vx1, qz1, kz1, vz1 = _tc_map(
        f_enc, 8, [xp, zp],
        [Wex, bex, Wez, bez, tx["q"][0], tx["q"][1], tx["k"][0], tx["k"][1],
         tx["v"][0], tx["v"][1], tz["q"][0], tz["q"][1], tz["k"][0],
         tz["k"][1], tz["v"][0], tz["v"][1]], 2000)
    kvx1 = jnp.concatenate([kx1, vx1], axis=1)
    kvz1 = jnp.concatenate([kz1, vz1], axis=1)

    def f_ea(e, W1, b1, W2, b2):
        return (e @ W1 + b1) @ W2 + b2
    ea = _tc_map(f_ea, 1, [eap], [We1, be1, We2, be2], 2000)

    tr_e = _make_tr(True)
    tr_ne = _make_tr(False)
    accx1 = tr_e(exs, exd, qx1, kvx1, ea)
    accz1 = tr_ne(z1s, z1d, qz1, kvz1)

    def f_comb_prep(a0, a1, xin, Ws, bs, Wq, bq, Wk, bk, Wv, bv):
        xn = jax.nn.relu(_norm((a0, a1), xin @ Ws + bs))
        return xn, xn @ Wq + bq, xn @ Wk + bk, xn @ Wv + bv
    tx2 = tr_w["convx2"]
    x1, qx2, kx2, vx2 = _tc_map(
        f_comb_prep, 4, [accx1[:N], accx1[N:], x0],
        [tx["skip"][0], tx["skip"][1], tx2["q"][0], tx2["q"][1],
         tx2["k"][0], tx2["k"][1], tx2["v"][0], tx2["v"][1]], 2000)
    kvx2 = jnp.concatenate([kx2, vx2], axis=1)
    accx2 = tr_e(exs, exd, qx2, kvx2, ea)

    tz2 = tr_w["convz2"]
    z1f, qz2, kz2, vz2 = _tc_map(
        f_comb_prep, 4, [accz1[:N], accz1[N:], z0],
        [tz["skip"][0], tz["skip"][1], tz2["q"][0], tz2["q"][1],
         tz2["k"][0], tz2["k"][1], tz2["v"][0], tz2["v"][1]], 2000)
    kvz2 = jnp.concatenate([kz2, vz2], axis=1)
    accz2 = tr_ne(z1s, z1d, qz2, kvz2)

    def f_xz(ax0, ax1, xin, az0, az1, zin, Wsx, bsx, Wsz, bsz,
             Wt, Wb, bxzv, Wm, bm):
        x2 = _norm((ax0, ax1), xin @ Wsx + bsx)
        z2 = _norm((az0, az1), zin @ Wsz + bsz)
        h0 = x2 @ Wt + z2 @ Wb + bxzv
        return h0, h0 @ Wm + bm
    h0, msg = _tc_map(
        f_xz, 2, [accx2[:N], accx2[N:], x1, accz2[:N], accz2[N:], z1f],
        [tx2["skip"][0], tx2["skip"][1], tz2["skip"][0], tz2["skip"][1],
         Wxz_t, Wxz_b, bxz, Wmsg, bmsg], 2000)

    accg = _gen_kernel(z1s, z1d, msg)

    def f_gen(a0, a1, hin, Wl, bl, Wr, br):
        h1 = jax.nn.relu(a0 + a1 + hin)
        return h1 @ Wl + bl, h1 @ Wr + br
    g2w = gat_w["convxz2"]
    xl2, xr2 = _tc_map(f_gen, 2, [accg[:N], accg[N:], h0],
                       [g2w[0], g2w[1], g2w[2], g2w[3]], 2000)

    att2_v = jnp.zeros((16,), f32).at[:10].set(p["convxz2"]["att"])
    accg2 = _gat_kernel(z2s, z2d, xl2, xr2, att2_v)

    def f_gat_comb(a0, a1, xl, xr, att, bias, Wl, bl, Wr, br):
        h = _gat_out((a0, a1), xl, xr, att, bias)
        return h @ Wl + bl, h @ Wr + br
    g3w = gat_w["convxz3"]
    xl3, xr3 = _tc_map(
        f_gat_comb, 2, [accg2[:N], accg2[N:], xl2, xr2],
        [g2w[4], g2w[5], g3w[0], g3w[1], g3w[2], g3w[3]], 2000)

    att3_v = jnp.zeros((16,), f32).at[:10].set(p["convxz3"]["att"])
    accg3 = _gat_kernel(z1s, z1d, xl3, xr3, att3_v)

    def f_gat3(a0, a1, xl, xr, att, bias):
        return _gat_out((a0, a1), xl, xr, att, bias)
    g3 = gat_w["convxz3"]
    h3 = _tc_map(f_gat3, 1, [accg3[:N], accg3[N:], xl3, xr3],
                 [g3[4], g3[5]], 2000)

    maxacc = _sage_kernel(z3s, z3d, h3)

    def f_sage(m0, m1, hin, Wl, bl, Wr, Wl5, bl5, Wr5, br5):
        agg = jnp.maximum(m0, m1)
        h4 = jax.nn.relu(agg @ Wl + bl + hin @ Wr)
        return h4 @ Wl5 + bl5, h4 @ Wr5 + br5
    g5w = gat_w["convxz5"]
    xl5, xr5 = _tc_map(f_sage, 2, [maxacc[:N], maxacc[N:], h3],
                       [Wl4, bl4, Wr4, g5w[0], g5w[1], g5w[2], g5w[3]], 2000)

    att5_v = jnp.zeros((16,), f32).at[:10].set(p["convxz5"]["att"])
    accg5 = _gat_kernel(z1s, z1d, xl5, xr5, att5_v)

    def f_final(a0, a1, xl, xr, att, bias, Wf, bf):
        h5 = _gat_out((a0, a1), xl, xr, att, bias)
        return h5 @ Wf + bf
    logits = _tc_map(f_final, 1, [accg5[:N], accg5[N:], xl5, xr5],
                     [g5w[4], g5w[5], Wlin, blin], 2000)

    rows = _pick_kernel(pickable, logits)

    def f_soft(r):
        l0, l1 = r[:, 0:1], r[:, 1:2]
        m = jnp.maximum(l0, l1)
        e0 = jnp.exp(l0 - m)
        e1 = jnp.exp(l1 - m)
        s = e0 + e1
        return jnp.concatenate(
            [e0 / s, e1 / s, jnp.zeros((r.shape[0], 14), f32)], axis=1)
    soft = _tc_map(f_soft, 1, [rows], [], 2048)
    return soft[:, :2]


# speedup vs baseline: 24.0579x; 1.1543x over previous
import functools
import math

import jax
import jax.numpy as jnp
from jax import lax
from jax.experimental import pallas as pl
from jax.experimental.pallas import tpu as pltpu
from jax.experimental.pallas import tpu_sc as plsc

N = 100000
E = 1600000
NPICK = 16384
R = 16
NC, NS = 2, 16
NW = NC * NS
CHUNK = 1024
NCHUNKS = E // CHUNK
TAIL = E - NCHUNKS * CHUNK
CH_FULL, CH_REM = NCHUNKS // NW, NCHUNKS % NW
RPS = 6248
RPS_LAST = N - 15 * RPS
ZCH = 104
NZC = 60
SCALE = 1.0 / math.sqrt(10.0)

SG_CHUNK = 2048
SG_FULL = E // SG_CHUNK
SG_HALF = 391
OWN = 6256
OWN_LAST = N - 15 * OWN
ACC_ROWS = OWN + 16

_mesh = plsc.VectorSubcoreMesh(core_axis_name="c", subcore_axis_name="s",
                               num_cores=NC, num_subcores=NS)
_SC_PARAMS = pltpu.CompilerParams(needs_layout_passes=False,
                                  use_tc_tiling_on_sc=False)


def _iota16():
    return lax.iota(jnp.int32, 16)


def _full16(v):
    return jnp.full((16,), v, jnp.int32)


def _zero_rows(ref, n):
    def zr(i, _):
        ref[i] = jnp.zeros((16,), jnp.float32)
        return 0
    lax.fori_loop(0, n, zr, 0)


def _worker_ids():
    c = lax.axis_index("c")
    s = lax.axis_index("s")
    return c, s, s * NC + c


def _zero_acc(acc, zbuf, s):
    _zero_rows(zbuf, ZCH)
    base = s * RPS
    for j in range(NZC):
        pltpu.sync_copy(zbuf, acc.at[pl.ds(base + j * ZCH, ZCH)])
    t0 = base + NZC * ZCH

    @pl.when(s < 15)
    def _():
        pltpu.sync_copy(zbuf.at[pl.ds(0, RPS - NZC * ZCH)],
                        acc.at[pl.ds(t0, RPS - NZC * ZCH)])

    @pl.when(s == 15)
    def _():
        pltpu.sync_copy(zbuf.at[pl.ds(0, RPS_LAST - NZC * ZCH)],
                        acc.at[pl.ds(t0, RPS_LAST - NZC * ZCH)])


def _edge_loop(gw, body, tail_body):
    nch = CH_FULL + (gw < CH_REM).astype(jnp.int32)

    def outer(i, _):
        body(gw + i * NW)
        return 0
    lax.fori_loop(0, nch, outer, 0)

    @pl.when(gw == 31)
    def _():
        tail_body(NCHUNKS)


def _writeback(acc, out, c, s):
    plsc.subcore_barrier()
    base = c * N + s * RPS

    @pl.when(s < 15)
    def _():
        pltpu.sync_copy(acc.at[pl.ds(s * RPS, RPS)],
                        out.at[pl.ds(base, RPS)])

    @pl.when(s == 15)
    def _():
        pltpu.sync_copy(acc.at[pl.ds(15 * RPS, RPS_LAST)],
                        out.at[pl.ds(c * N + 15 * RPS, RPS_LAST)])


def _tr_body(has_e, *refs):
    if has_e:
        (src, dst, q_t, kv_t, ea_t, out, idx_s, idx_d,
         qr0, qr1, kvr0, kvr1, er0, er1, outr0, outr1,
         zbuf, acc, sg0, sg1, ss0, ss1) = refs
        er = [er0, er1]
    else:
        (src, dst, q_t, kv_t, out, idx_s, idx_d,
         qr0, qr1, kvr0, kvr1, outr0, outr1,
         zbuf, acc, sg0, sg1, ss0, ss1) = refs
        er = [None, None]
    qr, kvr, outr = [qr0, qr1], [kvr0, kvr1], [outr0, outr1]
    sg, ss = [sg0, sg1], [ss0, ss1]
    c, s, gw = _worker_ids()
    _zero_rows(outr0, 128)
    _zero_rows(outr1, 128)
    _zero_acc(acc, zbuf, s)
    plsc.subcore_barrier()
    iota = _iota16()

    def make_chunk(nrows):
        def chunk(cid):
            icps = [pltpu.async_copy(src.at[pl.ds(cid * 8, nrows)],
                                     idx_s.at[pl.ds(0, nrows)], sg0),
                    pltpu.async_copy(dst.at[pl.ds(cid * 8, nrows)],
                                     idx_d.at[pl.ds(0, nrows)], sg1)]
            for cp in icps:
                cp.wait()

            def fire(j):
                b = j & 1
                cps = [pltpu.async_copy(kv_t.at[idx_s.at[j]], kvr[b], sg[b]),
                       pltpu.async_copy(q_t.at[idx_d.at[j]], qr[b], sg[b])]
                if has_e:
                    cps.append(pltpu.async_copy(
                        ea_t.at[pl.ds(cid * CHUNK + j * 128, 128)],
                        er[b], sg[b]))
                return cps

            def compute(b):
                def grp(g, _):
                    ridx = iota + g * 16
                    score = jnp.zeros((16,), jnp.float32)
                    e_cols = []
                    for d in range(10):
                        qd = plsc.load_gather(qr[b], [ridx, _full16(d)])
                        kd = plsc.load_gather(kvr[b], [ridx, _full16(d)])
                        if has_e:
                            ed = plsc.load_gather(er[b], [ridx, _full16(d)])
                            e_cols.append(ed)
                            kd = kd + ed
                        score = score + qd * kd
                    ee = jnp.exp(score * SCALE)
                    for d in range(10):
                        vd = plsc.load_gather(kvr[b], [ridx, _full16(16 + d)])
                        if has_e:
                            vd = vd + e_cols[d]
                        plsc.store_scatter(outr[b], [ridx, _full16(d)],
                                           ee * vd)
                    plsc.store_scatter(outr[b], [ridx, _full16(10)], ee)
                    return 0
                lax.fori_loop(0, 8, grp, 0)

            gh = {0: fire(0)}
            sh = {}
            for j in range(nrows):
                b = j & 1
                for cp in gh.pop(j):
                    cp.wait()
                if j + 1 < nrows:
                    gh[j + 1] = fire(j + 1)
                if j - 2 in sh:
                    sh.pop(j - 2).wait()
                compute(b)
                sh[j] = pltpu.async_copy(outr[b], acc.at[idx_d.at[j]],
                                         ss[b], add=True)
            for h in sh.values():
                h.wait()
        return chunk

    _edge_loop(gw, make_chunk(8), make_chunk(4))
    _writeback(acc, out, c, s)


def _make_tr(has_e):
    scratch = [
        pltpu.VMEM((8, 128), jnp.int32),
        pltpu.VMEM((8, 128), jnp.int32),
        pltpu.VMEM((128, 16), jnp.float32),
        pltpu.VMEM((128, 16), jnp.float32),
        pltpu.VMEM((128, 32), jnp.float32),
        pltpu.VMEM((128, 32), jnp.float32),
    ]
    if has_e:
        scratch += [pltpu.VMEM((128, 16), jnp.float32),
                    pltpu.VMEM((128, 16), jnp.float32)]
    scratch += [
        pltpu.VMEM((128, 16), jnp.float32),
        pltpu.VMEM((128, 16), jnp.float32),
        pltpu.VMEM((ZCH, 16), jnp.float32),
        pltpu.VMEM_SHARED((N, 16), jnp.float32),
        pltpu.SemaphoreType.DMA,
        pltpu.SemaphoreType.DMA,
        pltpu.SemaphoreType.DMA,
        pltpu.SemaphoreType.DMA,
    ]
    return pl.kernel(
        functools.partial(_tr_body, has_e),
        out_type=jax.ShapeDtypeStruct((NC * N, 16), jnp.float32),
        mesh=_mesh,
        compiler_params=_SC_PARAMS,
        scratch_types=scratch,
    )


def _gat_body(src, dst, xl_t, xr_t, att_t, out,
              idx_s, idx_d, xlr0, xlr1, xrr0, xrr1, outr0, outr1,
              attv, zbuf, acc, sg0, sg1, ss0, ss1):
    xlr, xrr, outr = [xlr0, xlr1], [xrr0, xrr1], [outr0, outr1]
    sg, ss = [sg0, sg1], [ss0, ss1]
    c, s, gw = _worker_ids()
    _zero_rows(outr0, 128)
    _zero_rows(outr1, 128)
    _zero_acc(acc, zbuf, s)
    pltpu.sync_copy(att_t, attv)
    plsc.subcore_barrier()
    iota = _iota16()
    att_cols = [plsc.load_gather(attv, [_full16(d)]) for d in range(10)]

    def make_chunk(nrows):
        def chunk(cid):
            icps = [pltpu.async_copy(src.at[pl.ds(cid * 8, nrows)],
                                     idx_s.at[pl.ds(0, nrows)], sg0),
                    pltpu.async_copy(dst.at[pl.ds(cid * 8, nrows)],
                                     idx_d.at[pl.ds(0, nrows)], sg1)]
            for cp in icps:
                cp.wait()

            def fire(j):
                b = j & 1
                return [pltpu.async_copy(xl_t.at[idx_s.at[j]], xlr[b],
                                         sg[b]),
                        pltpu.async_copy(xr_t.at[idx_d.at[j]], xrr[b],
                                         sg[b])]

            def compute(b):
                def grp(g, _):
                    ridx = iota + g * 16
                    score = jnp.zeros((16,), jnp.float32)
                    l_cols = []
                    for d in range(10):
                        ld = plsc.load_gather(xlr[b], [ridx, _full16(d)])
                        rd = plsc.load_gather(xrr[b], [ridx, _full16(d)])
                        m = ld + rd
                        m = jnp.where(m >= 0.0, m, m * 0.2)
                        score = score + m * att_cols[d]
                        l_cols.append(ld)
                    ee = jnp.exp(score)
                    for d in range(10):
                        plsc.store_scatter(outr[b], [ridx, _full16(d)],
                                           ee * l_cols[d])
                    plsc.store_scatter(outr[b], [ridx, _full16(10)], ee)
                    return 0
                lax.fori_loop(0, 8, grp, 0)

            gh = {0: fire(0)}
            sh = {}
            for j in range(nrows):
                b = j & 1
                for cp in gh.pop(j):
                    cp.wait()
                if j + 1 < nrows:
                    gh[j + 1] = fire(j + 1)
                if j - 2 in sh:
                    sh.pop(j - 2).wait()
                compute(b)
                sh[j] = pltpu.async_copy(outr[b], acc.at[idx_d.at[j]],
                                         ss[b], add=True)
            for h in sh.values():
                h.wait()
        return chunk

    _edge_loop(gw, make_chunk(8), make_chunk(4))
    _writeback(acc, out, c, s)


_gat_kernel = pl.kernel(
    _gat_body,
    out_type=jax.ShapeDtypeStruct((NC * N, 16), jnp.float32),
    mesh=_mesh,
    compiler_params=_SC_PARAMS,
    scratch_types=[
        pltpu.VMEM((8, 128), jnp.int32),
        pltpu.VMEM((8, 128), jnp.int32),
        pltpu.VMEM((128, 16), jnp.float32),
        pltpu.VMEM((128, 16), jnp.float32),
        pltpu.VMEM((128, 16), jnp.float32),
        pltpu.VMEM((128, 16), jnp.float32),
        pltpu.VMEM((128, 16), jnp.float32),
        pltpu.VMEM((128, 16), jnp.float32),
        pltpu.VMEM((16,), jnp.float32),
        pltpu.VMEM((ZCH, 16), jnp.float32),
        pltpu.VMEM_SHARED((N, 16), jnp.float32),
        pltpu.SemaphoreType.DMA,
        pltpu.SemaphoreType.DMA,
        pltpu.SemaphoreType.DMA,
        pltpu.SemaphoreType.DMA,
    ],
)


def _gen_body(src, dst, msg_t, out, idx_s, idx_d, outr0, outr1,
              zbuf, acc, sg0, sg1, ss0, ss1):
    outr = [outr0, outr1]
    sg, ss = [sg0, sg1], [ss0, ss1]
    c, s, gw = _worker_ids()
    _zero_acc(acc, zbuf, s)
    plsc.subcore_barrier()

    def make_chunk(nrows):
        def chunk(cid):
            icps = [pltpu.async_copy(src.at[pl.ds(cid * 8, nrows)],
                                     idx_s.at[pl.ds(0, nrows)], sg0),
                    pltpu.async_copy(dst.at[pl.ds(cid * 8, nrows)],
                                     idx_d.at[pl.ds(0, nrows)], sg1)]
            for cp in icps:
                cp.wait()

            def fire(j):
                b = j & 1
                return pltpu.async_copy(msg_t.at[idx_s.at[j]], outr[b],
                                        sg[b])

            gh = {0: fire(0)}
            sh = {}
            for j in range(nrows):
                b = j & 1
                gh.pop(j).wait()
                sh[j] = pltpu.async_copy(outr[b], acc.at[idx_d.at[j]],
                                         ss[b], add=True)
                if j + 1 < nrows:
                    if j - 1 in sh:
                        sh.pop(j - 1).wait()
                    gh[j + 1] = fire(j + 1)
            for h in sh.values():
                h.wait()
        return chunk

    _edge_loop(gw, make_chunk(8), make_chunk(4))
    _writeback(acc, out, c, s)


_gen_kernel = pl.kernel(
    _gen_body,
    out_type=jax.ShapeDtypeStruct((NC * N, 16), jnp.float32),
    mesh=_mesh,
    compiler_params=_SC_PARAMS,
    scratch_types=[
        pltpu.VMEM((8, 128), jnp.int32),
        pltpu.VMEM((8, 128), jnp.int32),
        pltpu.VMEM((128, 16), jnp.float32),
        pltpu.VMEM((128, 16), jnp.float32),
        pltpu.VMEM((ZCH, 16), jnp.float32),
        pltpu.VMEM_SHARED((N, 16), jnp.float32),
        pltpu.SemaphoreType.DMA,
        pltpu.SemaphoreType.DMA,
        pltpu.SemaphoreType.DMA,
        pltpu.SemaphoreType.DMA,
    ],
)


def _sage_body(src, dst, x_t, out,
               sbuf, dbuf, stag_s, stag_d, rbuf, acc, sem):
    c, s, gw = _worker_ids()
    lo = s * OWN
    iota = _iota16()
    _zero_rows(acc, ACC_ROWS)

    def scan_chunk(row0, nrows):
        cps = [pltpu.async_copy(src.at[pl.ds(row0, nrows)],
                                sbuf.at[pl.ds(0, nrows)], sem),
               pltpu.async_copy(dst.at[pl.ds(row0, nrows)],
                                dbuf.at[pl.ds(0, nrows)], sem)]
        for cp in cps:
            cp.wait()
        ngroups = nrows * 8

        def grp(g, wp):
            rr = _full16(g >> 3)
            cc = (g & 7) * 16 + iota
            dv = plsc.load_gather(dbuf, [rr, cc])
            sv = plsc.load_gather(sbuf, [rr, cc])
            own = (dv >= lo) & (dv < lo + OWN)
            pc = plsc.cumsum(jnp.where(own, 1, 0))
            pos = wp + pc - 1
            plsc.store_scatter(stag_s, [pos], sv, mask=own)
            plsc.store_scatter(stag_d, [pos], dv - lo, mask=own)
            cnt = plsc.all_reduce_population_count(own)
            return wp + cnt[0]
        wp = lax.fori_loop(0, ngroups, grp, 0)
        for j in range(8):
            pidx = wp + iota + 16 * j
            plsc.store_scatter(stag_s, [pidx], _full16(0))
            plsc.store_scatter(stag_d, [pidx], _full16(OWN))
        nb = (wp + 127) >> 7

        def batch(b, _):
            cp = pltpu.async_copy(
                x_t.at[stag_s.at[pl.ds(b * 128, 128)]], rbuf, sem)
            cp.wait()

            def g2(g, _):
                dl = plsc.load_gather(stag_d, [b * 128 + g * 16 + iota])
                for l in range(16):
                    di = _full16(dl[l])
                    row = plsc.load_gather(rbuf, [_full16(g * 16 + l), iota])
                    old = plsc.load_gather(acc, [di, iota])
                    plsc.store_scatter(acc, [di, iota],
                                       jnp.maximum(old, row))
                return 0
            lax.fori_loop(0, 8, g2, 0)
            return 0
        lax.fori_loop(0, nb, batch, 0)

    half = jnp.where(c == 0, 0, SG_HALF)
    nfull = jnp.where(c == 0, SG_HALF, SG_FULL - SG_HALF)

    def outer(i, _):
        scan_chunk((half + i) * 16, 16)
        return 0
    lax.fori_loop(0, nfull, outer, 0)

    @pl.when(c == 1)
    def _():
        scan_chunk(SG_FULL * 16, 4)
    base = c * N + lo

    @pl.when(s < 15)
    def _():
        pltpu.sync_copy(acc.at[pl.ds(0, OWN)], out.at[pl.ds(base, OWN)])

    @pl.when(s == 15)
    def _():
        pltpu.sync_copy(acc.at[pl.ds(0, OWN_LAST)],
                        out.at[pl.ds(base, OWN_LAST)])


_sage_kernel = pl.kernel(
    _sage_body,
    out_type=jax.ShapeDtypeStruct((NC * N, 16), jnp.float32),
    mesh=_mesh,
    compiler_params=_SC_PARAMS,
    scratch_types=[
        pltpu.VMEM((16, 128), jnp.int32),
        pltpu.VMEM((16, 128), jnp.int32),
        pltpu.VMEM((SG_CHUNK + 128,), jnp.int32),
        pltpu.VMEM((SG_CHUNK + 128,), jnp.int32),
        pltpu.VMEM((128, 16), jnp.float32),
        pltpu.VMEM((ACC_ROWS, 16), jnp.float32),
        pltpu.SemaphoreType.DMA,
    ],
)


def _pick_body(pick, logit_t, out, idxb, rbuf, sem):
    c, s, gw = _worker_ids()
    pltpu.sync_copy(pick.at[pl.ds(gw * 512, 512)], idxb)
    cps = []
    for j in range(4):
        cps.append(pltpu.async_copy(
            logit_t.at[idxb.at[pl.ds(j * 128, 128)]],
            rbuf.at[pl.ds(j * 128, 128)], sem))
    for cp in cps:
        cp.wait()
    pltpu.sync_copy(rbuf, out.at[pl.ds(gw * 512, 512)])


_pick_kernel = pl.kernel(
    _pick_body,
    out_type=jax.ShapeDtypeStruct((NPICK, 16), jnp.float32),
    mesh=_mesh,
    compiler_params=_SC_PARAMS,
    scratch_types=[
        pltpu.VMEM((512,), jnp.int32),
        pltpu.VMEM((512, 16), jnp.float32),
        pltpu.SemaphoreType.DMA,
    ],
)


def _tc_map(fn, out_cols, arrays, weights, block_rows, rows):
    grid = rows // block_rows
    ents = [(a, 0) if not isinstance(a, tuple) else a for a in arrays]
    in_specs = [pl.BlockSpec((block_rows, a.shape[1]),
                             lambda i, o=off: (i + o, 0))
                for a, off in ents]
    in_specs += [pl.BlockSpec(w.shape, lambda i: (0,) * w.ndim)
                 for w in weights]
    na = len(ents)
    n_out = len(out_cols)

    def body(*refs):
        ins = [r[...] for r in refs[:na + len(weights)]]
        outs = refs[na + len(weights):]
        res = fn(*ins)
        if n_out == 1:
            res = (res,)
        for o, v in zip(outs, res):
            o[...] = v
    out_shape = [jax.ShapeDtypeStruct((rows, cc), jnp.float32)
                 for cc in out_cols]
    out_specs = [pl.BlockSpec((block_rows, cc), lambda i: (i, 0))
                 for cc in out_cols]
    res = pl.pallas_call(
        body, grid=(grid,), in_specs=in_specs,
        out_specs=out_specs if n_out > 1 else out_specs[0],
        out_shape=out_shape if n_out > 1 else out_shape[0],
    )(*[a for a, _ in ents], *weights)
    return res


def _pad_w(p, din=16, dout=16):
    W = p["W"]
    out = jnp.zeros((din, dout), jnp.float32)
    out = out.at[:W.shape[0], :W.shape[1]].set(W)
    b = jnp.zeros((1, dout), jnp.float32)
    if "b" in p:
        b = b.at[0, :p["b"].shape[0]].set(p["b"])
    return out, b


def _colmask():
    return (lax.broadcasted_iota(jnp.int32, (1, 16), 1) < 10).astype(
        jnp.float32)


def _norm(acc2, skip):
    acc = acc2[0] + acc2[1]
    den = acc[:, 10:11]
    return acc * _colmask() / (den + 1e-16) + skip


def _leaky_self(xl, xr, att):
    m = xl + xr
    m = jnp.where(m >= 0.0, m, m * 0.2)
    es = jnp.exp(jnp.sum(m * att, axis=1, keepdims=True))
    return es


def _gat_out(acc2, xl, xr, att, bias):
    acc = acc2[0] + acc2[1]
    es = _leaky_self(xl, xr, att)
    num = acc * _colmask() + es * xl
    den = acc[:, 10:11] + es
    return jax.nn.relu(num / (den + 1e-16) + bias)


def kernel(x, z, edge_index, z1edge_index, z2edge_index, z3edge_index,
           edge_attr, pickable, params):
    p = params
    f32 = jnp.float32

    exs = edge_index[0].reshape(E // 128, 128)
    exd = edge_index[1].reshape(E // 128, 128)
    z1s = z1edge_index[0].reshape(E // 128, 128)
    z1d = z1edge_index[1].reshape(E // 128, 128)
    z2s = z2edge_index[0].reshape(E // 128, 128)
    z2d = z2edge_index[1].reshape(E // 128, 128)
    z3s = z3edge_index[0].reshape(E // 128, 128)
    z3d = z3edge_index[1].reshape(E // 128, 128)

    Wex, bex = _pad_w(p["encx"], 3)
    Wez, bez = _pad_w(p["encz"], 4)
    We1, be1 = _pad_w(p["edge1"], 6)
    We2, be2 = _pad_w(p["edge2"])
    tr_w = {}
    for name in ("convx1", "convx2", "convz1", "convz2"):
        tp = p[name]
        tr_w[name] = {k: _pad_w(tp[k]) for k in tp}
    _, bxz = _pad_w(p["linxz"], 32)
    Wxz_t = jnp.zeros((16, 16), f32).at[:10, :10].set(p["linxz"]["W"][:10])
    Wxz_b = jnp.zeros((16, 16), f32).at[:10, :10].set(p["linxz"]["W"][10:])
    Wmsg, bmsg = _pad_w(p["convxz1"]["msg"])
    gat_w = {}
    for name in ("convxz2", "convxz3", "convxz5"):
        gp = p[name]
        Wl, bl = _pad_w(gp["l"])
        Wr, br = _pad_w(gp["r"])
        att = jnp.zeros((1, 16), f32).at[0, :10].set(gp["att"])
        bias = jnp.zeros((1, 16), f32).at[0, :10].set(gp["bias"])
        gat_w[name] = (Wl, bl, Wr, br, att, bias)
    Wl4, bl4 = _pad_w(p["convxz4"]["l"])
    Wr4, _ = _pad_w(p["convxz4"]["r"])
    Wlin, blin = _pad_w(p["lin"])

    def f_enc(xb, zb, Wx, bx, Wz, bz, Wq, bq, Wk, bk, Wv, bv,
              Wqz, bqz, Wkz, bkz, Wvz, bvz):
        x0 = jax.nn.relu(xb @ Wx + bx)
        z0 = zb @ Wz + bz
        return (x0, z0, x0 @ Wq + bq,
                jnp.concatenate([x0 @ Wk + bk, x0 @ Wv + bv], axis=1),
                z0 @ Wqz + bqz,
                jnp.concatenate([z0 @ Wkz + bkz, z0 @ Wvz + bvz], axis=1))
    tx = tr_w["convx1"]
    tz = tr_w["convz1"]
    x0, z0, qx1, kvx1, qz1, kvz1 = _tc_map(
        f_enc, [16, 16, 16, 32, 16, 32], [x, z],
        [Wex, bex, Wez, bez, tx["q"][0], tx["q"][1], tx["k"][0], tx["k"][1],
         tx["v"][0], tx["v"][1], tz["q"][0], tz["q"][1], tz["k"][0],
         tz["k"][1], tz["v"][0], tz["v"][1]], 2000, N)

    def f_ea(e, W1, b1, W2, b2):
        return (e @ W1 + b1) @ W2 + b2
    ea = _tc_map(f_ea, [16], [edge_attr], [We1, be1, We2, be2], 2000, E)

    tr_e = _make_tr(True)
    tr_ne = _make_tr(False)
    accx1 = tr_e(exs, exd, qx1, kvx1, ea)
    accz1 = tr_ne(z1s, z1d, qz1, kvz1)

    def f_comb_prep(a0, a1, xin, Ws, bs, Wq, bq, Wk, bk, Wv, bv):
        xn = jax.nn.relu(_norm((a0, a1), xin @ Ws + bs))
        return (xn, xn @ Wq + bq,
                jnp.concatenate([xn @ Wk + bk, xn @ Wv + bv], axis=1))
    tx2 = tr_w["convx2"]
    x1, qx2, kvx2 = _tc_map(
        f_comb_prep, [16, 16, 32], [accx1, (accx1, 50), x0],
        [tx["skip"][0], tx["skip"][1], tx2["q"][0], tx2["q"][1],
         tx2["k"][0], tx2["k"][1], tx2["v"][0], tx2["v"][1]], 2000, N)
    accx2 = tr_e(exs, exd, qx2, kvx2, ea)

    tz2 = tr_w["convz2"]
    z1f, qz2, kvz2 = _tc_map(
        f_comb_prep, [16, 16, 32], [accz1, (accz1, 50), z0],
        [tz["skip"][0], tz["skip"][1], tz2["q"][0], tz2["q"][1],
         tz2["k"][0], tz2["k"][1], tz2["v"][0], tz2["v"][1]], 2000, N)
    accz2 = tr_ne(z1s, z1d, qz2, kvz2)

    def f_xz(ax0, ax1, xin, az0, az1, zin, Wsx, bsx, Wsz, bsz,
             Wt, Wb, bxzv, Wm, bm):
        x2 = _norm((ax0, ax1), xin @ Wsx + bsx)
        z2 = _norm((az0, az1), zin @ Wsz + bsz)
        h0 = x2 @ Wt + z2 @ Wb + bxzv
        return h0, h0 @ Wm + bm
    h0, msg = _tc_map(
        f_xz, [16, 16],
        [accx2, (accx2, 50), x1, accz2, (accz2, 50), z1f],
        [tx2["skip"][0], tx2["skip"][1], tz2["skip"][0], tz2["skip"][1],
         Wxz_t, Wxz_b, bxz, Wmsg, bmsg], 2000, N)

    accg = _gen_kernel(z1s, z1d, msg)

    def f_gen(a0, a1, hin, Wl, bl, Wr, br):
        h1 = jax.nn.relu(a0 + a1 + hin)
        return h1 @ Wl + bl, h1 @ Wr + br
    g2w = gat_w["convxz2"]
    xl2, xr2 = _tc_map(f_gen, [16, 16], [accg, (accg, 50), h0],
                       [g2w[0], g2w[1], g2w[2], g2w[3]], 2000, N)

    att2_v = jnp.zeros((16,), f32).at[:10].set(p["convxz2"]["att"])
    accg2 = _gat_kernel(z2s, z2d, xl2, xr2, att2_v)

    def f_gat_comb(a0, a1, xl, xr, att, bias, Wl, bl, Wr, br):
        h = _gat_out((a0, a1), xl, xr, att, bias)
        return h @ Wl + bl, h @ Wr + br
    g3w = gat_w["convxz3"]
    xl3, xr3 = _tc_map(
        f_gat_comb, [16, 16], [accg2, (accg2, 50), xl2, xr2],
        [g2w[4], g2w[5], g3w[0], g3w[1], g3w[2], g3w[3]], 2000, N)

    att3_v = jnp.zeros((16,), f32).at[:10].set(p["convxz3"]["att"])
    accg3 = _gat_kernel(z1s, z1d, xl3, xr3, att3_v)

    def f_gat3(a0, a1, xl, xr, att, bias):
        return _gat_out((a0, a1), xl, xr, att, bias)
    g3 = gat_w["convxz3"]
    h3 = _tc_map(f_gat3, [16], [accg3, (accg3, 50), xl3, xr3],
                 [g3[4], g3[5]], 2000, N)

    maxacc = _sage_kernel(z3s, z3d, h3)

    def f_sage(m0, m1, hin, Wl, bl, Wr, Wl5, bl5, Wr5, br5):
        agg = jnp.maximum(m0, m1)
        h4 = jax.nn.relu(agg @ Wl + bl + hin @ Wr)
        return h4 @ Wl5 + bl5, h4 @ Wr5 + br5
    g5w = gat_w["convxz5"]
    xl5, xr5 = _tc_map(f_sage, [16, 16], [maxacc, (maxacc, 50), h3],
                       [Wl4, bl4, Wr4, g5w[0], g5w[1], g5w[2], g5w[3]],
                       2000, N)

    att5_v = jnp.zeros((16,), f32).at[:10].set(p["convxz5"]["att"])
    accg5 = _gat_kernel(z1s, z1d, xl5, xr5, att5_v)

    def f_final(a0, a1, xl, xr, att, bias, Wf, bf):
        h5 = _gat_out((a0, a1), xl, xr, att, bias)
        return h5 @ Wf + bf
    logits = _tc_map(f_final, [16], [accg5, (accg5, 50), xl5, xr5],
                     [g5w[4], g5w[5], Wlin, blin], 2000, N)

    rows = _pick_kernel(pickable, logits)

    def f_soft(r):
        l0, l1 = r[:, 0:1], r[:, 1:2]
        m = jnp.maximum(l0, l1)
        e0 = jnp.exp(l0 - m)
        e1 = jnp.exp(l1 - m)
        s = e0 + e1
        return jnp.concatenate([e0 / s, e1 / s], axis=1)
    return _tc_map(f_soft, [2], [rows], [], 2048, NPICK)


# speedup vs baseline: 24.0790x; 1.0009x over previous
import functools
import math

import jax
import jax.numpy as jnp
from jax import lax
from jax.experimental import pallas as pl
from jax.experimental.pallas import tpu as pltpu
from jax.experimental.pallas import tpu_sc as plsc

N = 100000
E = 1600000
NPICK = 16384
R = 16
NC, NS = 2, 16
NW = NC * NS
CHUNK = 1024
NCHUNKS = E // CHUNK
TAIL = E - NCHUNKS * CHUNK
CH_FULL, CH_REM = NCHUNKS // NW, NCHUNKS % NW
RPS = 6248
RPS_LAST = N - 15 * RPS
ZCH = 104
NZC = 60
SCALE = 1.0 / math.sqrt(10.0)

SG_CHUNK = 2048
SG_FULL = E // SG_CHUNK
SG_HALF = 391
OWN = 6256
OWN_LAST = N - 15 * OWN
ACC_ROWS = OWN + 16

_mesh = plsc.VectorSubcoreMesh(core_axis_name="c", subcore_axis_name="s",
                               num_cores=NC, num_subcores=NS)
_SC_PARAMS = pltpu.CompilerParams(needs_layout_passes=False,
                                  use_tc_tiling_on_sc=False)


def _iota16():
    return lax.iota(jnp.int32, 16)


def _full16(v):
    return jnp.full((16,), v, jnp.int32)


def _zero_rows(ref, n):
    def zr(i, _):
        ref[i] = jnp.zeros((16,), jnp.float32)
        return 0
    lax.fori_loop(0, n, zr, 0)


def _worker_ids():
    c = lax.axis_index("c")
    s = lax.axis_index("s")
    return c, s, s * NC + c


def _zero_acc(acc, zbuf, s):
    _zero_rows(zbuf, ZCH)
    base = s * RPS
    for j in range(NZC):
        pltpu.sync_copy(zbuf, acc.at[pl.ds(base + j * ZCH, ZCH)])
    t0 = base + NZC * ZCH

    @pl.when(s < 15)
    def _():
        pltpu.sync_copy(zbuf.at[pl.ds(0, RPS - NZC * ZCH)],
                        acc.at[pl.ds(t0, RPS - NZC * ZCH)])

    @pl.when(s == 15)
    def _():
        pltpu.sync_copy(zbuf.at[pl.ds(0, RPS_LAST - NZC * ZCH)],
                        acc.at[pl.ds(t0, RPS_LAST - NZC * ZCH)])


def _edge_loop(gw, body, tail_body):
    nch = CH_FULL + (gw < CH_REM).astype(jnp.int32)

    def outer(i, _):
        body(gw + i * NW)
        return 0
    lax.fori_loop(0, nch, outer, 0)

    @pl.when(gw == 31)
    def _():
        tail_body(NCHUNKS)


def _writeback(acc, out, c, s):
    plsc.subcore_barrier()
    base = c * N + s * RPS

    @pl.when(s < 15)
    def _():
        pltpu.sync_copy(acc.at[pl.ds(s * RPS, RPS)],
                        out.at[pl.ds(base, RPS)])

    @pl.when(s == 15)
    def _():
        pltpu.sync_copy(acc.at[pl.ds(15 * RPS, RPS_LAST)],
                        out.at[pl.ds(c * N + 15 * RPS, RPS_LAST)])


def _tr_body(has_e, *refs):
    if has_e:
        (src, dst, q_t, kv_t, ea_t, out, idx_s, idx_d,
         qr0, qr1, kvr0, kvr1, er0, er1, outr0, outr1,
         zbuf, acc, sg0, sg1, ss0, ss1) = refs
        er = [er0, er1]
    else:
        (src, dst, q_t, kv_t, out, idx_s, idx_d,
         qr0, qr1, kvr0, kvr1, outr0, outr1,
         zbuf, acc, sg0, sg1, ss0, ss1) = refs
        er = [None, None]
    qr, kvr, outr = [qr0, qr1], [kvr0, kvr1], [outr0, outr1]
    sg, ss = [sg0, sg1], [ss0, ss1]
    c, s, gw = _worker_ids()
    _zero_rows(outr0, 128)
    _zero_rows(outr1, 128)
    _zero_acc(acc, zbuf, s)
    plsc.subcore_barrier()
    iota = _iota16()

    def make_chunk(nrows):
        def chunk(cid):
            icps = [pltpu.async_copy(src.at[pl.ds(cid * 8, nrows)],
                                     idx_s.at[pl.ds(0, nrows)], sg0),
                    pltpu.async_copy(dst.at[pl.ds(cid * 8, nrows)],
                                     idx_d.at[pl.ds(0, nrows)], sg1)]
            for cp in icps:
                cp.wait()

            def fire(j):
                b = j & 1
                cps = [pltpu.async_copy(kv_t.at[idx_s.at[j]], kvr[b], sg[b]),
                       pltpu.async_copy(q_t.at[idx_d.at[j]], qr[b], sg[b])]
                if has_e:
                    cps.append(pltpu.async_copy(
                        ea_t.at[pl.ds(cid * CHUNK + j * 128, 128)],
                        er[b], sg[b]))
                return cps

            def compute(b):
                def grp(g, _):
                    ridx = iota + g * 16
                    score = jnp.zeros((16,), jnp.float32)
                    e_cols = []
                    for d in range(10):
                        qd = plsc.load_gather(qr[b], [ridx, _full16(d)])
                        kd = plsc.load_gather(kvr[b], [ridx, _full16(d)])
                        if has_e:
                            ed = plsc.load_gather(er[b], [ridx, _full16(d)])
                            e_cols.append(ed)
                            kd = kd + ed
                        score = score + qd * kd
                    ee = jnp.exp(score * SCALE)
                    for d in range(10):
                        vd = plsc.load_gather(kvr[b], [ridx, _full16(16 + d)])
                        if has_e:
                            vd = vd + e_cols[d]
                        plsc.store_scatter(outr[b], [ridx, _full16(d)],
                                           ee * vd)
                    plsc.store_scatter(outr[b], [ridx, _full16(10)], ee)
                    return 0
                lax.fori_loop(0, 8, grp, 0)

            gh = {0: fire(0)}
            sh = {}
            for j in range(nrows):
                b = j & 1
                for cp in gh.pop(j):
                    cp.wait()
                if j + 1 < nrows:
                    gh[j + 1] = fire(j + 1)
                if j - 2 in sh:
                    sh.pop(j - 2).wait()
                compute(b)
                sh[j] = pltpu.async_copy(outr[b], acc.at[idx_d.at[j]],
                                         ss[b], add=True)
            for h in sh.values():
                h.wait()
        return chunk

    _edge_loop(gw, make_chunk(8), make_chunk(4))
    _writeback(acc, out, c, s)


def _make_tr(has_e):
    scratch = [
        pltpu.VMEM((8, 128), jnp.int32),
        pltpu.VMEM((8, 128), jnp.int32),
        pltpu.VMEM((128, 16), jnp.float32),
        pltpu.VMEM((128, 16), jnp.float32),
        pltpu.VMEM((128, 32), jnp.float32),
        pltpu.VMEM((128, 32), jnp.float32),
    ]
    if has_e:
        scratch += [pltpu.VMEM((128, 16), jnp.float32),
                    pltpu.VMEM((128, 16), jnp.float32)]
    scratch += [
        pltpu.VMEM((128, 16), jnp.float32),
        pltpu.VMEM((128, 16), jnp.float32),
        pltpu.VMEM((ZCH, 16), jnp.float32),
        pltpu.VMEM_SHARED((N, 16), jnp.float32),
        pltpu.SemaphoreType.DMA,
        pltpu.SemaphoreType.DMA,
        pltpu.SemaphoreType.DMA,
        pltpu.SemaphoreType.DMA,
    ]
    return pl.kernel(
        functools.partial(_tr_body, has_e),
        out_type=jax.ShapeDtypeStruct((NC * N, 16), jnp.float32),
        mesh=_mesh,
        compiler_params=_SC_PARAMS,
        scratch_types=scratch,
    )


def _gat_body(src, dst, xl_t, xr_t, att_t, out,
              idx_s, idx_d, xlr0, xlr1, xrr0, xrr1, outr0, outr1,
              attv, zbuf, acc, sg0, sg1, ss0, ss1):
    xlr, xrr, outr = [xlr0, xlr1], [xrr0, xrr1], [outr0, outr1]
    sg, ss = [sg0, sg1], [ss0, ss1]
    c, s, gw = _worker_ids()
    _zero_rows(outr0, 128)
    _zero_rows(outr1, 128)
    _zero_acc(acc, zbuf, s)
    pltpu.sync_copy(att_t, attv)
    plsc.subcore_barrier()
    iota = _iota16()
    att_cols = [plsc.load_gather(attv, [_full16(d)]) for d in range(10)]

    def make_chunk(nrows):
        def chunk(cid):
            icps = [pltpu.async_copy(src.at[pl.ds(cid * 8, nrows)],
                                     idx_s.at[pl.ds(0, nrows)], sg0),
                    pltpu.async_copy(dst.at[pl.ds(cid * 8, nrows)],
                                     idx_d.at[pl.ds(0, nrows)], sg1)]
            for cp in icps:
                cp.wait()

            def fire(j):
                b = j & 1
                return [pltpu.async_copy(xl_t.at[idx_s.at[j]], xlr[b],
                                         sg[b]),
                        pltpu.async_copy(xr_t.at[idx_d.at[j]], xrr[b],
                                         sg[b])]

            def compute(b):
                def grp(g, _):
                    ridx = iota + g * 16
                    score = jnp.zeros((16,), jnp.float32)
                    l_cols = []
                    for d in range(10):
                        ld = plsc.load_gather(xlr[b], [ridx, _full16(d)])
                        rd = plsc.load_gather(xrr[b], [ridx, _full16(d)])
                        m = ld + rd
                        m = jnp.where(m >= 0.0, m, m * 0.2)
                        score = score + m * att_cols[d]
                        l_cols.append(ld)
                    ee = jnp.exp(score)
                    for d in range(10):
                        plsc.store_scatter(outr[b], [ridx, _full16(d)],
                                           ee * l_cols[d])
                    plsc.store_scatter(outr[b], [ridx, _full16(10)], ee)
                    return 0
                lax.fori_loop(0, 8, grp, 0)

            gh = {0: fire(0)}
            sh = {}
            for j in range(nrows):
                b = j & 1
                for cp in gh.pop(j):
                    cp.wait()
                if j + 1 < nrows:
                    gh[j + 1] = fire(j + 1)
                if j - 2 in sh:
                    sh.pop(j - 2).wait()
                compute(b)
                sh[j] = pltpu.async_copy(outr[b], acc.at[idx_d.at[j]],
                                         ss[b], add=True)
            for h in sh.values():
                h.wait()
        return chunk

    _edge_loop(gw, make_chunk(8), make_chunk(4))
    _writeback(acc, out, c, s)


_gat_kernel = pl.kernel(
    _gat_body,
    out_type=jax.ShapeDtypeStruct((NC * N, 16), jnp.float32),
    mesh=_mesh,
    compiler_params=_SC_PARAMS,
    scratch_types=[
        pltpu.VMEM((8, 128), jnp.int32),
        pltpu.VMEM((8, 128), jnp.int32),
        pltpu.VMEM((128, 16), jnp.float32),
        pltpu.VMEM((128, 16), jnp.float32),
        pltpu.VMEM((128, 16), jnp.float32),
        pltpu.VMEM((128, 16), jnp.float32),
        pltpu.VMEM((128, 16), jnp.float32),
        pltpu.VMEM((128, 16), jnp.float32),
        pltpu.VMEM((16,), jnp.float32),
        pltpu.VMEM((ZCH, 16), jnp.float32),
        pltpu.VMEM_SHARED((N, 16), jnp.float32),
        pltpu.SemaphoreType.DMA,
        pltpu.SemaphoreType.DMA,
        pltpu.SemaphoreType.DMA,
        pltpu.SemaphoreType.DMA,
    ],
)


def _gen_body(src, dst, msg_t, out, idx_s, idx_d, outr0, outr1,
              zbuf, acc, sg0, sg1, ss0, ss1):
    outr = [outr0, outr1]
    sg, ss = [sg0, sg1], [ss0, ss1]
    c, s, gw = _worker_ids()
    _zero_acc(acc, zbuf, s)
    plsc.subcore_barrier()

    def make_chunk(nrows):
        def chunk(cid):
            icps = [pltpu.async_copy(src.at[pl.ds(cid * 8, nrows)],
                                     idx_s.at[pl.ds(0, nrows)], sg0),
                    pltpu.async_copy(dst.at[pl.ds(cid * 8, nrows)],
                                     idx_d.at[pl.ds(0, nrows)], sg1)]
            for cp in icps:
                cp.wait()

            def fire(j):
                b = j & 1
                return pltpu.async_copy(msg_t.at[idx_s.at[j]], outr[b],
                                        sg[b])

            gh = {0: fire(0)}
            sh = {}
            for j in range(nrows):
                b = j & 1
                gh.pop(j).wait()
                sh[j] = pltpu.async_copy(outr[b], acc.at[idx_d.at[j]],
                                         ss[b], add=True)
                if j + 1 < nrows:
                    if j - 1 in sh:
                        sh.pop(j - 1).wait()
                    gh[j + 1] = fire(j + 1)
            for h in sh.values():
                h.wait()
        return chunk

    _edge_loop(gw, make_chunk(8), make_chunk(4))
    _writeback(acc, out, c, s)


_gen_kernel = pl.kernel(
    _gen_body,
    out_type=jax.ShapeDtypeStruct((NC * N, 16), jnp.float32),
    mesh=_mesh,
    compiler_params=_SC_PARAMS,
    scratch_types=[
        pltpu.VMEM((8, 128), jnp.int32),
        pltpu.VMEM((8, 128), jnp.int32),
        pltpu.VMEM((128, 16), jnp.float32),
        pltpu.VMEM((128, 16), jnp.float32),
        pltpu.VMEM((ZCH, 16), jnp.float32),
        pltpu.VMEM_SHARED((N, 16), jnp.float32),
        pltpu.SemaphoreType.DMA,
        pltpu.SemaphoreType.DMA,
        pltpu.SemaphoreType.DMA,
        pltpu.SemaphoreType.DMA,
    ],
)


def _sage_body(src, dst, x_t, out,
               sbuf, dbuf, stag_s, stag_d, rbuf, acc, sem):
    c, s, gw = _worker_ids()
    lo = s * OWN
    iota = _iota16()
    _zero_rows(acc, ACC_ROWS)

    def scan_chunk(row0, nrows):
        cps = [pltpu.async_copy(src.at[pl.ds(row0, nrows)],
                                sbuf.at[pl.ds(0, nrows)], sem),
               pltpu.async_copy(dst.at[pl.ds(row0, nrows)],
                                dbuf.at[pl.ds(0, nrows)], sem)]
        for cp in cps:
            cp.wait()
        ngroups = nrows * 8

        def grp(g, wp):
            rr = _full16(g >> 3)
            cc = (g & 7) * 16 + iota
            dv = plsc.load_gather(dbuf, [rr, cc])
            sv = plsc.load_gather(sbuf, [rr, cc])
            own = (dv >= lo) & (dv < lo + OWN)
            pc = plsc.cumsum(jnp.where(own, 1, 0))
            pos = wp + pc - 1
            plsc.store_scatter(stag_s, [pos], sv, mask=own)
            plsc.store_scatter(stag_d, [pos], dv - lo, mask=own)
            cnt = plsc.all_reduce_population_count(own)
            return wp + cnt[0]
        wp = lax.fori_loop(0, ngroups, grp, 0)
        for j in range(8):
            pidx = wp + iota + 16 * j
            plsc.store_scatter(stag_s, [pidx], _full16(0))
            plsc.store_scatter(stag_d, [pidx], _full16(OWN))
        nb = (wp + 127) >> 7

        def batch(b, _):
            cp = pltpu.async_copy(
                x_t.at[stag_s.at[pl.ds(b * 128, 128)]], rbuf, sem)
            cp.wait()

            def g2(g, _):
                @pl.when(b * 128 + g * 16 < wp)
                def _():
                    dl = plsc.load_gather(stag_d, [b * 128 + g * 16 + iota])
                    for l in range(16):
                        di = _full16(dl[l])
                        row = plsc.load_gather(rbuf,
                                               [_full16(g * 16 + l), iota])
                        old = plsc.load_gather(acc, [di, iota])
                        plsc.store_scatter(acc, [di, iota],
                                           jnp.maximum(old, row))
                return 0
            lax.fori_loop(0, 8, g2, 0)
            return 0
        lax.fori_loop(0, nb, batch, 0)

    half = jnp.where(c == 0, 0, SG_HALF)
    nfull = jnp.where(c == 0, SG_HALF, SG_FULL - SG_HALF)

    def outer(i, _):
        scan_chunk((half + i) * 16, 16)
        return 0
    lax.fori_loop(0, nfull, outer, 0)

    @pl.when(c == 1)
    def _():
        scan_chunk(SG_FULL * 16, 4)
    base = c * N + lo

    @pl.when(s < 15)
    def _():
        pltpu.sync_copy(acc.at[pl.ds(0, OWN)], out.at[pl.ds(base, OWN)])

    @pl.when(s == 15)
    def _():
        pltpu.sync_copy(acc.at[pl.ds(0, OWN_LAST)],
                        out.at[pl.ds(base, OWN_LAST)])


_sage_kernel = pl.kernel(
    _sage_body,
    out_type=jax.ShapeDtypeStruct((NC * N, 16), jnp.float32),
    mesh=_mesh,
    compiler_params=_SC_PARAMS,
    scratch_types=[
        pltpu.VMEM((16, 128), jnp.int32),
        pltpu.VMEM((16, 128), jnp.int32),
        pltpu.VMEM((SG_CHUNK + 128,), jnp.int32),
        pltpu.VMEM((SG_CHUNK + 128,), jnp.int32),
        pltpu.VMEM((128, 16), jnp.float32),
        pltpu.VMEM((ACC_ROWS, 16), jnp.float32),
        pltpu.SemaphoreType.DMA,
    ],
)


def _pick_body(pick, logit_t, out, idxb, rbuf, sem):
    c, s, gw = _worker_ids()
    pltpu.sync_copy(pick.at[pl.ds(gw * 512, 512)], idxb)
    cps = []
    for j in range(4):
        cps.append(pltpu.async_copy(
            logit_t.at[idxb.at[pl.ds(j * 128, 128)]],
            rbuf.at[pl.ds(j * 128, 128)], sem))
    for cp in cps:
        cp.wait()
    pltpu.sync_copy(rbuf, out.at[pl.ds(gw * 512, 512)])


_pick_kernel = pl.kernel(
    _pick_body,
    out_type=jax.ShapeDtypeStruct((NPICK, 16), jnp.float32),
    mesh=_mesh,
    compiler_params=_SC_PARAMS,
    scratch_types=[
        pltpu.VMEM((512,), jnp.int32),
        pltpu.VMEM((512, 16), jnp.float32),
        pltpu.SemaphoreType.DMA,
    ],
)


def _tc_map(fn, out_cols, arrays, weights, block_rows, rows):
    grid = rows // block_rows
    ents = [(a, 0) if not isinstance(a, tuple) else a for a in arrays]
    in_specs = [pl.BlockSpec((block_rows, a.shape[1]),
                             lambda i, o=off: (i + o, 0))
                for a, off in ents]
    in_specs += [pl.BlockSpec(w.shape, lambda i: (0,) * w.ndim)
                 for w in weights]
    na = len(ents)
    n_out = len(out_cols)

    def body(*refs):
        ins = [r[...] for r in refs[:na + len(weights)]]
        outs = refs[na + len(weights):]
        res = fn(*ins)
        if n_out == 1:
            res = (res,)
        for o, v in zip(outs, res):
            o[...] = v
    out_shape = [jax.ShapeDtypeStruct((rows, cc), jnp.float32)
                 for cc in out_cols]
    out_specs = [pl.BlockSpec((block_rows, cc), lambda i: (i, 0))
                 for cc in out_cols]
    res = pl.pallas_call(
        body, grid=(grid,), in_specs=in_specs,
        out_specs=out_specs if n_out > 1 else out_specs[0],
        out_shape=out_shape if n_out > 1 else out_shape[0],
    )(*[a for a, _ in ents], *weights)
    return res


def _pad_w(p, din=16, dout=16):
    W = p["W"]
    out = jnp.zeros((din, dout), jnp.float32)
    out = out.at[:W.shape[0], :W.shape[1]].set(W)
    b = jnp.zeros((1, dout), jnp.float32)
    if "b" in p:
        b = b.at[0, :p["b"].shape[0]].set(p["b"])
    return out, b


def _colmask():
    return (lax.broadcasted_iota(jnp.int32, (1, 16), 1) < 10).astype(
        jnp.float32)


def _norm(acc2, skip):
    acc = acc2[0] + acc2[1]
    den = acc[:, 10:11]
    return acc * _colmask() / (den + 1e-16) + skip


def _leaky_self(xl, xr, att):
    m = xl + xr
    m = jnp.where(m >= 0.0, m, m * 0.2)
    es = jnp.exp(jnp.sum(m * att, axis=1, keepdims=True))
    return es


def _gat_out(acc2, xl, xr, att, bias):
    acc = acc2[0] + acc2[1]
    es = _leaky_self(xl, xr, att)
    num = acc * _colmask() + es * xl
    den = acc[:, 10:11] + es
    return jax.nn.relu(num / (den + 1e-16) + bias)


def kernel(x, z, edge_index, z1edge_index, z2edge_index, z3edge_index,
           edge_attr, pickable, params):
    p = params
    f32 = jnp.float32

    exs = edge_index[0].reshape(E // 128, 128)
    exd = edge_index[1].reshape(E // 128, 128)
    z1s = z1edge_index[0].reshape(E // 128, 128)
    z1d = z1edge_index[1].reshape(E // 128, 128)
    z2s = z2edge_index[0].reshape(E // 128, 128)
    z2d = z2edge_index[1].reshape(E // 128, 128)
    z3s = z3edge_index[0].reshape(E // 128, 128)
    z3d = z3edge_index[1].reshape(E // 128, 128)

    Wex, bex = _pad_w(p["encx"], 3)
    Wez, bez = _pad_w(p["encz"], 4)
    We1, be1 = _pad_w(p["edge1"], 6)
    We2, be2 = _pad_w(p["edge2"])
    tr_w = {}
    for name in ("convx1", "convx2", "convz1", "convz2"):
        tp = p[name]
        tr_w[name] = {k: _pad_w(tp[k]) for k in tp}
    _, bxz = _pad_w(p["linxz"], 32)
    Wxz_t = jnp.zeros((16, 16), f32).at[:10, :10].set(p["linxz"]["W"][:10])
    Wxz_b = jnp.zeros((16, 16), f32).at[:10, :10].set(p["linxz"]["W"][10:])
    Wmsg, bmsg = _pad_w(p["convxz1"]["msg"])
    gat_w = {}
    for name in ("convxz2", "convxz3", "convxz5"):
        gp = p[name]
        Wl, bl = _pad_w(gp["l"])
        Wr, br = _pad_w(gp["r"])
        att = jnp.zeros((1, 16), f32).at[0, :10].set(gp["att"])
        bias = jnp.zeros((1, 16), f32).at[0, :10].set(gp["bias"])
        gat_w[name] = (Wl, bl, Wr, br, att, bias)
    Wl4, bl4 = _pad_w(p["convxz4"]["l"])
    Wr4, _ = _pad_w(p["convxz4"]["r"])
    Wlin, blin = _pad_w(p["lin"])

    def f_enc(xb, zb, Wx, bx, Wz, bz, Wq, bq, Wk, bk, Wv, bv,
              Wqz, bqz, Wkz, bkz, Wvz, bvz):
        x0 = jax.nn.relu(xb @ Wx + bx)
        z0 = zb @ Wz + bz
        return (x0, z0, x0 @ Wq + bq,
                jnp.concatenate([x0 @ Wk + bk, x0 @ Wv + bv], axis=1),
                z0 @ Wqz + bqz,
                jnp.concatenate([z0 @ Wkz + bkz, z0 @ Wvz + bvz], axis=1))
    tx = tr_w["convx1"]
    tz = tr_w["convz1"]
    x0, z0, qx1, kvx1, qz1, kvz1 = _tc_map(
        f_enc, [16, 16, 16, 32, 16, 32], [x, z],
        [Wex, bex, Wez, bez, tx["q"][0], tx["q"][1], tx["k"][0], tx["k"][1],
         tx["v"][0], tx["v"][1], tz["q"][0], tz["q"][1], tz["k"][0],
         tz["k"][1], tz["v"][0], tz["v"][1]], 2000, N)

    def f_ea(e, W1, b1, W2, b2):
        return (e @ W1 + b1) @ W2 + b2
    ea = _tc_map(f_ea, [16], [edge_attr], [We1, be1, We2, be2], 2000, E)

    tr_e = _make_tr(True)
    tr_ne = _make_tr(False)
    accx1 = tr_e(exs, exd, qx1, kvx1, ea)
    accz1 = tr_ne(z1s, z1d, qz1, kvz1)

    def f_comb_prep(a0, a1, xin, Ws, bs, Wq, bq, Wk, bk, Wv, bv):
        xn = jax.nn.relu(_norm((a0, a1), xin @ Ws + bs))
        return (xn, xn @ Wq + bq,
                jnp.concatenate([xn @ Wk + bk, xn @ Wv + bv], axis=1))
    tx2 = tr_w["convx2"]
    x1, qx2, kvx2 = _tc_map(
        f_comb_prep, [16, 16, 32], [accx1, (accx1, 50), x0],
        [tx["skip"][0], tx["skip"][1], tx2["q"][0], tx2["q"][1],
         tx2["k"][0], tx2["k"][1], tx2["v"][0], tx2["v"][1]], 2000, N)
    accx2 = tr_e(exs, exd, qx2, kvx2, ea)

    tz2 = tr_w["convz2"]
    z1f, qz2, kvz2 = _tc_map(
        f_comb_prep, [16, 16, 32], [accz1, (accz1, 50), z0],
        [tz["skip"][0], tz["skip"][1], tz2["q"][0], tz2["q"][1],
         tz2["k"][0], tz2["k"][1], tz2["v"][0], tz2["v"][1]], 2000, N)
    accz2 = tr_ne(z1s, z1d, qz2, kvz2)

    def f_xz(ax0, ax1, xin, az0, az1, zin, Wsx, bsx, Wsz, bsz,
             Wt, Wb, bxzv, Wm, bm):
        x2 = _norm((ax0, ax1), xin @ Wsx + bsx)
        z2 = _norm((az0, az1), zin @ Wsz + bsz)
        h0 = x2 @ Wt + z2 @ Wb + bxzv
        return h0, h0 @ Wm + bm
    h0, msg = _tc_map(
        f_xz, [16, 16],
        [accx2, (accx2, 50), x1, accz2, (accz2, 50), z1f],
        [tx2["skip"][0], tx2["skip"][1], tz2["skip"][0], tz2["skip"][1],
         Wxz_t, Wxz_b, bxz, Wmsg, bmsg], 2000, N)

    accg = _gen_kernel(z1s, z1d, msg)

    def f_gen(a0, a1, hin, Wl, bl, Wr, br):
        h1 = jax.nn.relu(a0 + a1 + hin)
        return h1 @ Wl + bl, h1 @ Wr + br
    g2w = gat_w["convxz2"]
    xl2, xr2 = _tc_map(f_gen, [16, 16], [accg, (accg, 50), h0],
                       [g2w[0], g2w[1], g2w[2], g2w[3]], 2000, N)

    att2_v = jnp.zeros((16,), f32).at[:10].set(p["convxz2"]["att"])
    accg2 = _gat_kernel(z2s, z2d, xl2, xr2, att2_v)

    def f_gat_comb(a0, a1, xl, xr, att, bias, Wl, bl, Wr, br):
        h = _gat_out((a0, a1), xl, xr, att, bias)
        return h @ Wl + bl, h @ Wr + br
    g3w = gat_w["convxz3"]
    xl3, xr3 = _tc_map(
        f_gat_comb, [16, 16], [accg2, (accg2, 50), xl2, xr2],
        [g2w[4], g2w[5], g3w[0], g3w[1], g3w[2], g3w[3]], 2000, N)

    att3_v = jnp.zeros((16,), f32).at[:10].set(p["convxz3"]["att"])
    accg3 = _gat_kernel(z1s, z1d, xl3, xr3, att3_v)

    def f_gat3(a0, a1, xl, xr, att, bias):
        return _gat_out((a0, a1), xl, xr, att, bias)
    g3 = gat_w["convxz3"]
    h3 = _tc_map(f_gat3, [16], [accg3, (accg3, 50), xl3, xr3],
                 [g3[4], g3[5]], 2000, N)

    maxacc = _sage_kernel(z3s, z3d, h3)

    def f_sage(m0, m1, hin, Wl, bl, Wr, Wl5, bl5, Wr5, br5):
        agg = jnp.maximum(m0, m1)
        h4 = jax.nn.relu(agg @ Wl + bl + hin @ Wr)
        return h4 @ Wl5 + bl5, h4 @ Wr5 + br5
    g5w = gat_w["convxz5"]
    xl5, xr5 = _tc_map(f_sage, [16, 16], [maxacc, (maxacc, 50), h3],
                       [Wl4, bl4, Wr4, g5w[0], g5w[1], g5w[2], g5w[3]],
                       2000, N)

    att5_v = jnp.zeros((16,), f32).at[:10].set(p["convxz5"]["att"])
    accg5 = _gat_kernel(z1s, z1d, xl5, xr5, att5_v)

    def f_final(a0, a1, xl, xr, att, bias, Wf, bf):
        h5 = _gat_out((a0, a1), xl, xr, att, bias)
        return h5 @ Wf + bf
    logits = _tc_map(f_final, [16], [accg5, (accg5, 50), xl5, xr5],
                     [g5w[4], g5w[5], Wlin, blin], 2000, N)

    rows = _pick_kernel(pickable, logits)

    def f_soft(r):
        l0, l1 = r[:, 0:1], r[:, 1:2]
        m = jnp.maximum(l0, l1)
        e0 = jnp.exp(l0 - m)
        e1 = jnp.exp(l1 - m)
        s = e0 + e1
        return jnp.concatenate([e0 / s, e1 / s], axis=1)
    return _tc_map(f_soft, [2], [rows], [], 2048, NPICK)


# speedup vs baseline: 28.1654x; 1.1697x over previous
import functools
import math

import jax
import jax.numpy as jnp
from jax import lax
from jax.experimental import pallas as pl
from jax.experimental.pallas import tpu as pltpu
from jax.experimental.pallas import tpu_sc as plsc

N = 100000
E = 1600000
NPICK = 16384
R = 16
NC, NS = 2, 16
NW = NC * NS
CHUNK = 1024
NCHUNKS = E // CHUNK
TAIL = E - NCHUNKS * CHUNK
CH_FULL, CH_REM = NCHUNKS // NW, NCHUNKS % NW
RPS = 6248
RPS_LAST = N - 15 * RPS
ZCH = 104
NZC = 60
SCALE = 1.0 / math.sqrt(10.0)

SG_ROWS0 = 6248
SG_PAIRS = 195
OWN = 6256
OWN_LAST = N - 15 * OWN
ACC_ROWS = OWN + 16

_mesh = plsc.VectorSubcoreMesh(core_axis_name="c", subcore_axis_name="s",
                               num_cores=NC, num_subcores=NS)
_SC_PARAMS = pltpu.CompilerParams(needs_layout_passes=False,
                                  use_tc_tiling_on_sc=False)


def _iota16():
    return lax.iota(jnp.int32, 16)


def _full16(v):
    return jnp.full((16,), v, jnp.int32)


def _zero_rows(ref, n):
    def zr(i, _):
        ref[i] = jnp.zeros((16,), jnp.float32)
        return 0
    lax.fori_loop(0, n, zr, 0)


def _worker_ids():
    c = lax.axis_index("c")
    s = lax.axis_index("s")
    return c, s, s * NC + c


def _zero_acc(acc, zbuf, s):
    _zero_rows(zbuf, ZCH)
    base = s * RPS
    for j in range(NZC):
        pltpu.sync_copy(zbuf, acc.at[pl.ds(base + j * ZCH, ZCH)])
    t0 = base + NZC * ZCH

    @pl.when(s < 15)
    def _():
        pltpu.sync_copy(zbuf.at[pl.ds(0, RPS - NZC * ZCH)],
                        acc.at[pl.ds(t0, RPS - NZC * ZCH)])

    @pl.when(s == 15)
    def _():
        pltpu.sync_copy(zbuf.at[pl.ds(0, RPS_LAST - NZC * ZCH)],
                        acc.at[pl.ds(t0, RPS_LAST - NZC * ZCH)])


def _edge_loop(gw, body, tail_body):
    nch = CH_FULL + (gw < CH_REM).astype(jnp.int32)

    def outer(i, _):
        body(gw + i * NW)
        return 0
    lax.fori_loop(0, nch, outer, 0)

    @pl.when(gw == 31)
    def _():
        tail_body(NCHUNKS)


def _writeback(acc, out, c, s):
    plsc.subcore_barrier()
    base = c * N + s * RPS

    @pl.when(s < 15)
    def _():
        pltpu.sync_copy(acc.at[pl.ds(s * RPS, RPS)],
                        out.at[pl.ds(base, RPS)])

    @pl.when(s == 15)
    def _():
        pltpu.sync_copy(acc.at[pl.ds(15 * RPS, RPS_LAST)],
                        out.at[pl.ds(c * N + 15 * RPS, RPS_LAST)])


def _tr_body(has_e, *refs):
    if has_e:
        (src, dst, q_t, kv_t, ea_t, out, idx_s, idx_d,
         qr0, qr1, kvr0, kvr1, er0, er1, outr0, outr1,
         zbuf, acc, sg0, sg1, ss0, ss1) = refs
        er = [er0, er1]
    else:
        (src, dst, q_t, kv_t, out, idx_s, idx_d,
         qr0, qr1, kvr0, kvr1, outr0, outr1,
         zbuf, acc, sg0, sg1, ss0, ss1) = refs
        er = [None, None]
    qr, kvr, outr = [qr0, qr1], [kvr0, kvr1], [outr0, outr1]
    sg, ss = [sg0, sg1], [ss0, ss1]
    c, s, gw = _worker_ids()
    _zero_rows(outr0, 128)
    _zero_rows(outr1, 128)
    _zero_acc(acc, zbuf, s)
    plsc.subcore_barrier()
    iota = _iota16()

    def make_chunk(nrows):
        def chunk(cid):
            icps = [pltpu.async_copy(src.at[pl.ds(cid * 8, nrows)],
                                     idx_s.at[pl.ds(0, nrows)], sg0),
                    pltpu.async_copy(dst.at[pl.ds(cid * 8, nrows)],
                                     idx_d.at[pl.ds(0, nrows)], sg1)]
            for cp in icps:
                cp.wait()

            def fire(j):
                b = j & 1
                cps = [pltpu.async_copy(kv_t.at[idx_s.at[j]], kvr[b], sg[b]),
                       pltpu.async_copy(q_t.at[idx_d.at[j]], qr[b], sg[b])]
                if has_e:
                    cps.append(pltpu.async_copy(
                        ea_t.at[pl.ds(cid * CHUNK + j * 128, 128)],
                        er[b], sg[b]))
                return cps

            def compute(b):
                def grp(g, _):
                    ridx = iota + g * 16
                    score = jnp.zeros((16,), jnp.float32)
                    e_cols = []
                    for d in range(10):
                        qd = plsc.load_gather(qr[b], [ridx, _full16(d)])
                        kd = plsc.load_gather(kvr[b], [ridx, _full16(d)])
                        if has_e:
                            ed = plsc.load_gather(er[b], [ridx, _full16(d)])
                            e_cols.append(ed)
                            kd = kd + ed
                        score = score + qd * kd
                    ee = jnp.exp(score * SCALE)
                    for d in range(10):
                        vd = plsc.load_gather(kvr[b], [ridx, _full16(16 + d)])
                        if has_e:
                            vd = vd + e_cols[d]
                        plsc.store_scatter(outr[b], [ridx, _full16(d)],
                                           ee * vd)
                    plsc.store_scatter(outr[b], [ridx, _full16(10)], ee)
                    return 0
                lax.fori_loop(0, 8, grp, 0)

            gh = {0: fire(0)}
            sh = {}
            for j in range(nrows):
                b = j & 1
                for cp in gh.pop(j):
                    cp.wait()
                if j + 1 < nrows:
                    gh[j + 1] = fire(j + 1)
                if j - 2 in sh:
                    sh.pop(j - 2).wait()
                compute(b)
                sh[j] = pltpu.async_copy(outr[b], acc.at[idx_d.at[j]],
                                         ss[b], add=True)
            for h in sh.values():
                h.wait()
        return chunk

    _edge_loop(gw, make_chunk(8), make_chunk(4))
    _writeback(acc, out, c, s)


def _make_tr(has_e):
    scratch = [
        pltpu.VMEM((8, 128), jnp.int32),
        pltpu.VMEM((8, 128), jnp.int32),
        pltpu.VMEM((128, 16), jnp.float32),
        pltpu.VMEM((128, 16), jnp.float32),
        pltpu.VMEM((128, 32), jnp.float32),
        pltpu.VMEM((128, 32), jnp.float32),
    ]
    if has_e:
        scratch += [pltpu.VMEM((128, 16), jnp.float32),
                    pltpu.VMEM((128, 16), jnp.float32)]
    scratch += [
        pltpu.VMEM((128, 16), jnp.float32),
        pltpu.VMEM((128, 16), jnp.float32),
        pltpu.VMEM((ZCH, 16), jnp.float32),
        pltpu.VMEM_SHARED((N, 16), jnp.float32),
        pltpu.SemaphoreType.DMA,
        pltpu.SemaphoreType.DMA,
        pltpu.SemaphoreType.DMA,
        pltpu.SemaphoreType.DMA,
    ]
    return pl.kernel(
        functools.partial(_tr_body, has_e),
        out_type=jax.ShapeDtypeStruct((NC * N, 16), jnp.float32),
        mesh=_mesh,
        compiler_params=_SC_PARAMS,
        scratch_types=scratch,
    )


def _gat_body(src, dst, xl_t, xr_t, att_t, out,
              idx_s, idx_d, xlr0, xlr1, xrr0, xrr1, outr0, outr1,
              attv, zbuf, acc, sg0, sg1, ss0, ss1):
    xlr, xrr, outr = [xlr0, xlr1], [xrr0, xrr1], [outr0, outr1]
    sg, ss = [sg0, sg1], [ss0, ss1]
    c, s, gw = _worker_ids()
    _zero_rows(outr0, 128)
    _zero_rows(outr1, 128)
    _zero_acc(acc, zbuf, s)
    pltpu.sync_copy(att_t, attv)
    plsc.subcore_barrier()
    iota = _iota16()
    att_cols = [plsc.load_gather(attv, [_full16(d)]) for d in range(10)]

    def make_chunk(nrows):
        def chunk(cid):
            icps = [pltpu.async_copy(src.at[pl.ds(cid * 8, nrows)],
                                     idx_s.at[pl.ds(0, nrows)], sg0),
                    pltpu.async_copy(dst.at[pl.ds(cid * 8, nrows)],
                                     idx_d.at[pl.ds(0, nrows)], sg1)]
            for cp in icps:
                cp.wait()

            def fire(j):
                b = j & 1
                return [pltpu.async_copy(xl_t.at[idx_s.at[j]], xlr[b],
                                         sg[b]),
                        pltpu.async_copy(xr_t.at[idx_d.at[j]], xrr[b],
                                         sg[b])]

            def compute(b):
                def grp(g, _):
                    ridx = iota + g * 16
                    score = jnp.zeros((16,), jnp.float32)
                    l_cols = []
                    for d in range(10):
                        ld = plsc.load_gather(xlr[b], [ridx, _full16(d)])
                        rd = plsc.load_gather(xrr[b], [ridx, _full16(d)])
                        m = ld + rd
                        m = jnp.where(m >= 0.0, m, m * 0.2)
                        score = score + m * att_cols[d]
                        l_cols.append(ld)
                    ee = jnp.exp(score)
                    for d in range(10):
                        plsc.store_scatter(outr[b], [ridx, _full16(d)],
                                           ee * l_cols[d])
                    plsc.store_scatter(outr[b], [ridx, _full16(10)], ee)
                    return 0
                lax.fori_loop(0, 8, grp, 0)

            gh = {0: fire(0)}
            sh = {}
            for j in range(nrows):
                b = j & 1
                for cp in gh.pop(j):
                    cp.wait()
                if j + 1 < nrows:
                    gh[j + 1] = fire(j + 1)
                if j - 2 in sh:
                    sh.pop(j - 2).wait()
                compute(b)
                sh[j] = pltpu.async_copy(outr[b], acc.at[idx_d.at[j]],
                                         ss[b], add=True)
            for h in sh.values():
                h.wait()
        return chunk

    _edge_loop(gw, make_chunk(8), make_chunk(4))
    _writeback(acc, out, c, s)


_gat_kernel = pl.kernel(
    _gat_body,
    out_type=jax.ShapeDtypeStruct((NC * N, 16), jnp.float32),
    mesh=_mesh,
    compiler_params=_SC_PARAMS,
    scratch_types=[
        pltpu.VMEM((8, 128), jnp.int32),
        pltpu.VMEM((8, 128), jnp.int32),
        pltpu.VMEM((128, 16), jnp.float32),
        pltpu.VMEM((128, 16), jnp.float32),
        pltpu.VMEM((128, 16), jnp.float32),
        pltpu.VMEM((128, 16), jnp.float32),
        pltpu.VMEM((128, 16), jnp.float32),
        pltpu.VMEM((128, 16), jnp.float32),
        pltpu.VMEM((16,), jnp.float32),
        pltpu.VMEM((ZCH, 16), jnp.float32),
        pltpu.VMEM_SHARED((N, 16), jnp.float32),
        pltpu.SemaphoreType.DMA,
        pltpu.SemaphoreType.DMA,
        pltpu.SemaphoreType.DMA,
        pltpu.SemaphoreType.DMA,
    ],
)


def _gen_body(src, dst, msg_t, out, idx_s, idx_d, outr0, outr1,
              zbuf, acc, sg0, sg1, ss0, ss1):
    outr = [outr0, outr1]
    sg, ss = [sg0, sg1], [ss0, ss1]
    c, s, gw = _worker_ids()
    _zero_acc(acc, zbuf, s)
    plsc.subcore_barrier()

    def make_chunk(nrows):
        def chunk(cid):
            icps = [pltpu.async_copy(src.at[pl.ds(cid * 8, nrows)],
                                     idx_s.at[pl.ds(0, nrows)], sg0),
                    pltpu.async_copy(dst.at[pl.ds(cid * 8, nrows)],
                                     idx_d.at[pl.ds(0, nrows)], sg1)]
            for cp in icps:
                cp.wait()

            def fire(j):
                b = j & 1
                return pltpu.async_copy(msg_t.at[idx_s.at[j]], outr[b],
                                        sg[b])

            gh = {0: fire(0)}
            sh = {}
            for j in range(nrows):
                b = j & 1
                gh.pop(j).wait()
                sh[j] = pltpu.async_copy(outr[b], acc.at[idx_d.at[j]],
                                         ss[b], add=True)
                if j + 1 < nrows:
                    if j - 1 in sh:
                        sh.pop(j - 1).wait()
                    gh[j + 1] = fire(j + 1)
            for h in sh.values():
                h.wait()
        return chunk

    _edge_loop(gw, make_chunk(8), make_chunk(4))
    _writeback(acc, out, c, s)


_gen_kernel = pl.kernel(
    _gen_body,
    out_type=jax.ShapeDtypeStruct((NC * N, 16), jnp.float32),
    mesh=_mesh,
    compiler_params=_SC_PARAMS,
    scratch_types=[
        pltpu.VMEM((8, 128), jnp.int32),
        pltpu.VMEM((8, 128), jnp.int32),
        pltpu.VMEM((128, 16), jnp.float32),
        pltpu.VMEM((128, 16), jnp.float32),
        pltpu.VMEM((ZCH, 16), jnp.float32),
        pltpu.VMEM_SHARED((N, 16), jnp.float32),
        pltpu.SemaphoreType.DMA,
        pltpu.SemaphoreType.DMA,
        pltpu.SemaphoreType.DMA,
        pltpu.SemaphoreType.DMA,
    ],
)


def _sage_body(src, dst, x_t, out,
               sbuf, dbuf, stag_s, stag_d, rbuf, acc, sem):
    c, s, gw = _worker_ids()
    lo = s * OWN
    iota = _iota16()
    _zero_rows(acc, ACC_ROWS)

    def scan_chunk(row0, nrows):
        cps = [pltpu.async_copy(src.at[pl.ds(row0, nrows)],
                                sbuf.at[pl.ds(0, nrows)], sem),
               pltpu.async_copy(dst.at[pl.ds(row0, nrows)],
                                dbuf.at[pl.ds(0, nrows)], sem)]
        for cp in cps:
            cp.wait()
        ngroups = nrows * 8

        def grp(g, wp):
            rr = _full16(g >> 3)
            cc = (g & 7) * 16 + iota
            dv = plsc.load_gather(dbuf, [rr, cc])
            sv = plsc.load_gather(sbuf, [rr, cc])
            own = (dv >= lo) & (dv < lo + OWN)
            pc = plsc.cumsum(jnp.where(own, 1, 0))
            pos = wp + pc - 1
            plsc.store_scatter(stag_s, [pos], sv, mask=own)
            plsc.store_scatter(stag_d, [pos], dv - lo, mask=own)
            cnt = plsc.all_reduce_population_count(own)
            return wp + cnt[0]
        wp = lax.fori_loop(0, ngroups, grp, 0)
        for j in range(8):
            pidx = wp + iota + 16 * j
            plsc.store_scatter(stag_s, [pidx], _full16(0))
            plsc.store_scatter(stag_d, [pidx], _full16(OWN))
        nb = (wp + 127) >> 7

        def batch(b, _):
            cp = pltpu.async_copy(
                x_t.at[stag_s.at[pl.ds(b * 128, 128)]], rbuf, sem)
            cp.wait()

            def g2(g, _):
                @pl.when(b * 128 + g * 16 < wp)
                def _():
                    dl = plsc.load_gather(stag_d, [b * 128 + g * 16 + iota])
                    for l in range(16):
                        di = _full16(dl[l])
                        row = plsc.load_gather(rbuf,
                                               [_full16(g * 16 + l), iota])
                        old = plsc.load_gather(acc, [di, iota])
                        plsc.store_scatter(acc, [di, iota],
                                           jnp.maximum(old, row))
                return 0
            lax.fori_loop(0, 8, g2, 0)
            return 0
        lax.fori_loop(0, nb, batch, 0)

    half = c * SG_ROWS0

    def outer(i, _):
        scan_chunk(half + i * 32, 32)
        return 0
    lax.fori_loop(0, SG_PAIRS, outer, 0)

    @pl.when(c == 0)
    def _():
        scan_chunk(SG_PAIRS * 32, 8)

    @pl.when(c == 1)
    def _():
        scan_chunk(SG_ROWS0 + SG_PAIRS * 32, 12)
    base = c * N + lo

    @pl.when(s < 15)
    def _():
        pltpu.sync_copy(acc.at[pl.ds(0, OWN)], out.at[pl.ds(base, OWN)])

    @pl.when(s == 15)
    def _():
        pltpu.sync_copy(acc.at[pl.ds(0, OWN_LAST)],
                        out.at[pl.ds(base, OWN_LAST)])


_sage_kernel = pl.kernel(
    _sage_body,
    out_type=jax.ShapeDtypeStruct((NC * N, 16), jnp.float32),
    mesh=_mesh,
    compiler_params=_SC_PARAMS,
    scratch_types=[
        pltpu.VMEM((32, 128), jnp.int32),
        pltpu.VMEM((32, 128), jnp.int32),
        pltpu.VMEM((4096 + 128,), jnp.int32),
        pltpu.VMEM((4096 + 128,), jnp.int32),
        pltpu.VMEM((128, 16), jnp.float32),
        pltpu.VMEM((ACC_ROWS, 16), jnp.float32),
        pltpu.SemaphoreType.DMA,
    ],
)


def _pick_body(pick, logit_t, out, idxb, rbuf, sem):
    c, s, gw = _worker_ids()
    pltpu.sync_copy(pick.at[pl.ds(gw * 512, 512)], idxb)
    cps = []
    for j in range(4):
        cps.append(pltpu.async_copy(
            logit_t.at[idxb.at[pl.ds(j * 128, 128)]],
            rbuf.at[pl.ds(j * 128, 128)], sem))
    for cp in cps:
        cp.wait()
    pltpu.sync_copy(rbuf, out.at[pl.ds(gw * 512, 512)])


_pick_kernel = pl.kernel(
    _pick_body,
    out_type=jax.ShapeDtypeStruct((NPICK, 16), jnp.float32),
    mesh=_mesh,
    compiler_params=_SC_PARAMS,
    scratch_types=[
        pltpu.VMEM((512,), jnp.int32),
        pltpu.VMEM((512, 16), jnp.float32),
        pltpu.SemaphoreType.DMA,
    ],
)


def _tc_map(fn, out_cols, arrays, weights, block_rows, rows):
    grid = rows // block_rows
    ents = [(a, 0) if not isinstance(a, tuple) else a for a in arrays]
    in_specs = [pl.BlockSpec((block_rows, a.shape[1]),
                             lambda i, o=off: (i + o, 0))
                for a, off in ents]
    in_specs += [pl.BlockSpec(w.shape, lambda i: (0,) * w.ndim)
                 for w in weights]
    na = len(ents)
    n_out = len(out_cols)

    def body(*refs):
        ins = [r[...] for r in refs[:na + len(weights)]]
        outs = refs[na + len(weights):]
        res = fn(*ins)
        if n_out == 1:
            res = (res,)
        for o, v in zip(outs, res):
            o[...] = v
    out_shape = [jax.ShapeDtypeStruct((rows, cc), jnp.float32)
                 for cc in out_cols]
    out_specs = [pl.BlockSpec((block_rows, cc), lambda i: (i, 0))
                 for cc in out_cols]
    res = pl.pallas_call(
        body, grid=(grid,), in_specs=in_specs,
        out_specs=out_specs if n_out > 1 else out_specs[0],
        out_shape=out_shape if n_out > 1 else out_shape[0],
    )(*[a for a, _ in ents], *weights)
    return res


def _pad_w(p, din=16, dout=16):
    W = p["W"]
    out = jnp.zeros((din, dout), jnp.float32)
    out = out.at[:W.shape[0], :W.shape[1]].set(W)
    b = jnp.zeros((1, dout), jnp.float32)
    if "b" in p:
        b = b.at[0, :p["b"].shape[0]].set(p["b"])
    return out, b


def _colmask():
    return (lax.broadcasted_iota(jnp.int32, (1, 16), 1) < 10).astype(
        jnp.float32)


def _norm(acc2, skip):
    acc = acc2[0] + acc2[1]
    den = acc[:, 10:11]
    return acc * _colmask() / (den + 1e-16) + skip


def _leaky_self(xl, xr, att):
    m = xl + xr
    m = jnp.where(m >= 0.0, m, m * 0.2)
    es = jnp.exp(jnp.sum(m * att, axis=1, keepdims=True))
    return es


def _gat_out(acc2, xl, xr, att, bias):
    acc = acc2[0] + acc2[1]
    es = _leaky_self(xl, xr, att)
    num = acc * _colmask() + es * xl
    den = acc[:, 10:11] + es
    return jax.nn.relu(num / (den + 1e-16) + bias)


def kernel(x, z, edge_index, z1edge_index, z2edge_index, z3edge_index,
           edge_attr, pickable, params):
    p = params
    f32 = jnp.float32

    exs = edge_index[0].reshape(E // 128, 128)
    exd = edge_index[1].reshape(E // 128, 128)
    z1s = z1edge_index[0].reshape(E // 128, 128)
    z1d = z1edge_index[1].reshape(E // 128, 128)
    z2s = z2edge_index[0].reshape(E // 128, 128)
    z2d = z2edge_index[1].reshape(E // 128, 128)
    z3s = z3edge_index[0].reshape(E // 128, 128)
    z3d = z3edge_index[1].reshape(E // 128, 128)

    Wex, bex = _pad_w(p["encx"], 3)
    Wez, bez = _pad_w(p["encz"], 4)
    We1, be1 = _pad_w(p["edge1"], 6)
    We2, be2 = _pad_w(p["edge2"])
    tr_w = {}
    for name in ("convx1", "convx2", "convz1", "convz2"):
        tp = p[name]
        tr_w[name] = {k: _pad_w(tp[k]) for k in tp}
    _, bxz = _pad_w(p["linxz"], 32)
    Wxz_t = jnp.zeros((16, 16), f32).at[:10, :10].set(p["linxz"]["W"][:10])
    Wxz_b = jnp.zeros((16, 16), f32).at[:10, :10].set(p["linxz"]["W"][10:])
    Wmsg, bmsg = _pad_w(p["convxz1"]["msg"])
    gat_w = {}
    for name in ("convxz2", "convxz3", "convxz5"):
        gp = p[name]
        Wl, bl = _pad_w(gp["l"])
        Wr, br = _pad_w(gp["r"])
        att = jnp.zeros((1, 16), f32).at[0, :10].set(gp["att"])
        bias = jnp.zeros((1, 16), f32).at[0, :10].set(gp["bias"])
        gat_w[name] = (Wl, bl, Wr, br, att, bias)
    Wl4, bl4 = _pad_w(p["convxz4"]["l"])
    Wr4, _ = _pad_w(p["convxz4"]["r"])
    Wlin, blin = _pad_w(p["lin"])

    def f_enc(xb, zb, Wx, bx, Wz, bz, Wq, bq, Wk, bk, Wv, bv,
              Wqz, bqz, Wkz, bkz, Wvz, bvz):
        x0 = jax.nn.relu(xb @ Wx + bx)
        z0 = zb @ Wz + bz
        return (x0, z0, x0 @ Wq + bq,
                jnp.concatenate([x0 @ Wk + bk, x0 @ Wv + bv], axis=1),
                z0 @ Wqz + bqz,
                jnp.concatenate([z0 @ Wkz + bkz, z0 @ Wvz + bvz], axis=1))
    tx = tr_w["convx1"]
    tz = tr_w["convz1"]
    x0, z0, qx1, kvx1, qz1, kvz1 = _tc_map(
        f_enc, [16, 16, 16, 32, 16, 32], [x, z],
        [Wex, bex, Wez, bez, tx["q"][0], tx["q"][1], tx["k"][0], tx["k"][1],
         tx["v"][0], tx["v"][1], tz["q"][0], tz["q"][1], tz["k"][0],
         tz["k"][1], tz["v"][0], tz["v"][1]], 2000, N)

    def f_ea(e, W1, b1, W2, b2):
        return (e @ W1 + b1) @ W2 + b2
    ea = _tc_map(f_ea, [16], [edge_attr], [We1, be1, We2, be2], 2000, E)

    tr_e = _make_tr(True)
    tr_ne = _make_tr(False)
    accx1 = tr_e(exs, exd, qx1, kvx1, ea)
    accz1 = tr_ne(z1s, z1d, qz1, kvz1)

    def f_comb_prep(a0, a1, xin, Ws, bs, Wq, bq, Wk, bk, Wv, bv):
        xn = jax.nn.relu(_norm((a0, a1), xin @ Ws + bs))
        return (xn, xn @ Wq + bq,
                jnp.concatenate([xn @ Wk + bk, xn @ Wv + bv], axis=1))
    tx2 = tr_w["convx2"]
    x1, qx2, kvx2 = _tc_map(
        f_comb_prep, [16, 16, 32], [accx1, (accx1, 50), x0],
        [tx["skip"][0], tx["skip"][1], tx2["q"][0], tx2["q"][1],
         tx2["k"][0], tx2["k"][1], tx2["v"][0], tx2["v"][1]], 2000, N)
    accx2 = tr_e(exs, exd, qx2, kvx2, ea)

    tz2 = tr_w["convz2"]
    z1f, qz2, kvz2 = _tc_map(
        f_comb_prep, [16, 16, 32], [accz1, (accz1, 50), z0],
        [tz["skip"][0], tz["skip"][1], tz2["q"][0], tz2["q"][1],
         tz2["k"][0], tz2["k"][1], tz2["v"][0], tz2["v"][1]], 2000, N)
    accz2 = tr_ne(z1s, z1d, qz2, kvz2)

    def f_xz(ax0, ax1, xin, az0, az1, zin, Wsx, bsx, Wsz, bsz,
             Wt, Wb, bxzv, Wm, bm):
        x2 = _norm((ax0, ax1), xin @ Wsx + bsx)
        z2 = _norm((az0, az1), zin @ Wsz + bsz)
        h0 = x2 @ Wt + z2 @ Wb + bxzv
        return h0, h0 @ Wm + bm
    h0, msg = _tc_map(
        f_xz, [16, 16],
        [accx2, (accx2, 50), x1, accz2, (accz2, 50), z1f],
        [tx2["skip"][0], tx2["skip"][1], tz2["skip"][0], tz2["skip"][1],
         Wxz_t, Wxz_b, bxz, Wmsg, bmsg], 2000, N)

    accg = _gen_kernel(z1s, z1d, msg)

    def f_gen(a0, a1, hin, Wl, bl, Wr, br):
        h1 = jax.nn.relu(a0 + a1 + hin)
        return h1 @ Wl + bl, h1 @ Wr + br
    g2w = gat_w["convxz2"]
    xl2, xr2 = _tc_map(f_gen, [16, 16], [accg, (accg, 50), h0],
                       [g2w[0], g2w[1], g2w[2], g2w[3]], 2000, N)

    att2_v = jnp.zeros((16,), f32).at[:10].set(p["convxz2"]["att"])
    accg2 = _gat_kernel(z2s, z2d, xl2, xr2, att2_v)

    def f_gat_comb(a0, a1, xl, xr, att, bias, Wl, bl, Wr, br):
        h = _gat_out((a0, a1), xl, xr, att, bias)
        return h @ Wl + bl, h @ Wr + br
    g3w = gat_w["convxz3"]
    xl3, xr3 = _tc_map(
        f_gat_comb, [16, 16], [accg2, (accg2, 50), xl2, xr2],
        [g2w[4], g2w[5], g3w[0], g3w[1], g3w[2], g3w[3]], 2000, N)

    att3_v = jnp.zeros((16,), f32).at[:10].set(p["convxz3"]["att"])
    accg3 = _gat_kernel(z1s, z1d, xl3, xr3, att3_v)

    def f_gat3(a0, a1, xl, xr, att, bias):
        return _gat_out((a0, a1), xl, xr, att, bias)
    g3 = gat_w["convxz3"]
    h3 = _tc_map(f_gat3, [16], [accg3, (accg3, 50), xl3, xr3],
                 [g3[4], g3[5]], 2000, N)

    maxacc = _sage_kernel(z3s, z3d, h3)

    def f_sage(m0, m1, hin, Wl, bl, Wr, Wl5, bl5, Wr5, br5):
        agg = jnp.maximum(m0, m1)
        h4 = jax.nn.relu(agg @ Wl + bl + hin @ Wr)
        return h4 @ Wl5 + bl5, h4 @ Wr5 + br5
    g5w = gat_w["convxz5"]
    xl5, xr5 = _tc_map(f_sage, [16, 16], [maxacc, (maxacc, 50), h3],
                       [Wl4, bl4, Wr4, g5w[0], g5w[1], g5w[2], g5w[3]],
                       2000, N)

    att5_v = jnp.zeros((16,), f32).at[:10].set(p["convxz5"]["att"])
    accg5 = _gat_kernel(z1s, z1d, xl5, xr5, att5_v)

    def f_final(a0, a1, xl, xr, att, bias, Wf, bf):
        h5 = _gat_out((a0, a1), xl, xr, att, bias)
        return h5 @ Wf + bf
    logits = _tc_map(f_final, [16], [accg5, (accg5, 50), xl5, xr5],
                     [g5w[4], g5w[5], Wlin, blin], 2000, N)

    rows = _pick_kernel(pickable, logits)

    def f_soft(r):
        l0, l1 = r[:, 0:1], r[:, 1:2]
        m = jnp.maximum(l0, l1)
        e0 = jnp.exp(l0 - m)
        e1 = jnp.exp(l1 - m)
        s = e0 + e1
        return jnp.concatenate([e0 / s, e1 / s], axis=1)
    return _tc_map(f_soft, [2], [rows], [], 2048, NPICK)
